# Initial kernel scaffold; baseline (speedup 1.0000x reference)
import jax
import jax.numpy as jnp
from jax.experimental import pallas as pl


def kernel(x, edge_attr, params, edge_index, batch):
    raise NotImplementedError("write your pallas kernel here")


# speedup vs baseline: 9.3677x; 9.3677x over previous
import functools
import math

import jax
import jax.numpy as jnp
from jax import lax
from jax.experimental import pallas as pl
from jax.experimental.pallas import tpu as pltpu
from jax.experimental.pallas import tpu_sc as plsc

N = 10000
NP = 10240
E = 160000
EP = 163840
HID = 64
NF = 4
K1N = (N + 1) // 2
K2N = (K1N + 1) // 2
NW = 32
EPT = EP // NW
CH = 1024
NCH = EPT // CH
BE = 2048
NBE = EP // BE
NPS = NP // 16
_INV = 1.0 / math.sqrt(1.0 + 1e-5)

f32 = jnp.float32
i32 = jnp.int32


def _mesh():
    return plsc.VectorSubcoreMesh(core_axis_name="c", subcore_axis_name="s")


def _lrelu(x, s=0.01):
    return jnp.where(x >= 0, x, s * x)


@functools.lru_cache(maxsize=None)
def _build_k1():
    @functools.partial(
        pl.kernel,
        out_type=jax.ShapeDtypeStruct((8, EP), f32),
        mesh=_mesh(),
        compiler_params=pltpu.CompilerParams(needs_layout_passes=False, use_tc_tiling_on_sc=False),
        scratch_types=[
            pltpu.VMEM((NP * 4,), f32),
            pltpu.VMEM((EPT,), i32),
            pltpu.VMEM((4, EPT), f32),
        ],
    )
    def _k1(x_hbm, src_hbm, xs_hbm, xtab, sidx, xsb):
        cid = lax.axis_index("c")
        sid = lax.axis_index("s")
        wid = sid * 2 + cid
        base = pl.multiple_of(wid * EPT, 128)
        pltpu.sync_copy(x_hbm, xtab)
        pltpu.sync_copy(src_hbm.at[pl.ds(base, EPT)], sidx)

        def body(j, carry):
            iv = sidx[pl.ds(j * 16, 16)]
            iv4 = iv * 4
            for f in range(4):
                g = plsc.load_gather(xtab, [iv4 + f])
                xsb[f, pl.ds(j * 16, 16)] = g
            return carry

        lax.fori_loop(0, EPT // 16, body, 0)
        for f in range(4):
            pltpu.sync_copy(xsb.at[f], xs_hbm.at[f, pl.ds(base, EPT)])

    return _k1


def _k1_gather(xp4, srcp):
    return _build_k1()(xp4, srcp)


def _k2_body(a_ref, xs_ref, pk_ref, w2_ref, msg_ref, sm_ref):
    p = pl.program_id(0)
    i = pl.program_id(1)
    ab = a_ref[0].reshape(16 * 128)
    alpha = pk_ref[6, 0]
    beta = pk_ref[6, 1]
    pos = lax.broadcasted_iota(i32, (BE,), 0) + i * BE
    valid = pos < E
    s_l = _lrelu(alpha * ab + beta)

    @pl.when(p == 0)
    def _softmax_pass():
        mb = jnp.max(jnp.where(valid, s_l, -3e38))
        sb = jnp.sum(jnp.where(valid, jnp.exp(s_l - mb), 0.0))

        @pl.when(i == 0)
        def _init():
            sm_ref[0] = mb
            sm_ref[1] = sb

        @pl.when(i > 0)
        def _upd():
            m0 = sm_ref[0]
            s0 = sm_ref[1]
            mn = jnp.maximum(m0, mb)
            sm_ref[0] = mn
            sm_ref[1] = s0 * jnp.exp(m0 - mn) + sb * jnp.exp(mb - mn)

    @pl.when(p == 1)
    def _msg_pass():
        m0 = sm_ref[0]
        s0 = sm_ref[1]
        sm = jnp.exp(s_l - m0) / s0
        ea = _lrelu(ab * sm)
        A = pk_ref[0]
        C = pk_ref[1]
        h = _lrelu(ea[:, None] * A[None, :] + C[None, :])
        xs = xs_ref[...]
        acc = jnp.zeros((BE, HID), f32)
        for fdim in range(4):
            wf = w2_ref[:, fdim * HID:(fdim + 1) * HID]
            hf = jnp.dot(h, wf, preferred_element_type=f32) + pk_ref[2 + fdim][None, :]
            acc = acc + xs[fdim][:, None] * hf
        msg_ref[...] = acc


def _k2_msg(a3, xs, pk, w2):
    return pl.pallas_call(
        _k2_body,
        grid=(2, NBE),
        in_specs=[
            pl.BlockSpec((1, 16, 128), lambda p, i: (i, 0, 0)),
            pl.BlockSpec((8, BE), lambda p, i: (0, i)),
            pl.BlockSpec((8, HID), lambda p, i: (0, 0)),
            pl.BlockSpec((HID, NF * HID), lambda p, i: (0, 0)),
        ],
        out_specs=pl.BlockSpec((BE, HID), lambda p, i: (i, 0)),
        out_shape=jax.ShapeDtypeStruct((EP, HID), f32),
        scratch_shapes=[pltpu.SMEM((2,), f32)],
    )(a3, xs, pk, w2)


@functools.lru_cache(maxsize=None)
def _build_k3():
    @functools.partial(
        pl.kernel,
        out_type=[
            jax.ShapeDtypeStruct((2, NP, HID), f32),
            jax.ShapeDtypeStruct((2, NP, 16), f32),
        ],
        mesh=_mesh(),
        compiler_params=pltpu.CompilerParams(needs_layout_passes=False, use_tc_tiling_on_sc=False),
        scratch_types=[
            pltpu.VMEM_SHARED((NP, HID), f32),
            pltpu.VMEM_SHARED((NP, 16), f32),
            pltpu.VMEM((8, 128), i32),
            pltpu.VMEM((CH, HID), f32),
            pltpu.VMEM((128, 16), f32),
        ],
    )
    def _k3(msg_hbm, dst2d_hbm, z64_hbm, z16_hbm, ones_hbm,
            aggp_hbm, cntp_hbm, aggS, cntS, dbuf, mbuf, onesv):
        cid = lax.axis_index("c")
        sid = lax.axis_index("s")
        wid = sid * 2 + cid
        rs = pl.multiple_of(sid * NPS, 128)
        pltpu.sync_copy(z64_hbm.at[pl.ds(rs, NPS)], aggS.at[pl.ds(rs, NPS)])
        pltpu.sync_copy(z16_hbm.at[pl.ds(rs, NPS)], cntS.at[pl.ds(rs, NPS)])
        pltpu.sync_copy(ones_hbm, onesv)
        plsc.subcore_barrier()
        for ch in range(NCH):
            e0 = pl.multiple_of(wid * EPT + ch * CH, 128)
            r0 = pl.multiple_of(wid * (EPT // 128) + ch * (CH // 128), 8)
            pltpu.sync_copy(dst2d_hbm.at[pl.ds(r0, 8)], dbuf)
            pltpu.sync_copy(msg_hbm.at[pl.ds(e0, CH)], mbuf)
            for r in range(8):
                pltpu.sync_copy(mbuf.at[pl.ds(r * 128, 128)],
                                aggS.at[dbuf.at[r]], add=True)
                pltpu.sync_copy(onesv, cntS.at[dbuf.at[r]], add=True)
        plsc.subcore_barrier()
        pltpu.sync_copy(aggS.at[pl.ds(rs, NPS)], aggp_hbm.at[cid, pl.ds(rs, NPS)])
        pltpu.sync_copy(cntS.at[pl.ds(rs, NPS)], cntp_hbm.at[cid, pl.ds(rs, NPS)])

    return _k3


def _k3_scatter(msg, dst2d, z64, z16, ones16):
    return _build_k3()(msg, dst2d, z64, z16, ones16)


@functools.lru_cache(maxsize=None)
def _build_kseg():
    @functools.partial(
        pl.kernel,
        out_type=jax.ShapeDtypeStruct((2, NP, 16), f32),
        mesh=_mesh(),
        compiler_params=pltpu.CompilerParams(needs_layout_passes=False, use_tc_tiling_on_sc=False),
        scratch_types=[
            pltpu.VMEM_SHARED((NP, 16), f32),
            pltpu.VMEM_SHARED((NP, 16), f32),
            pltpu.VMEM((8, 128), i32),
            pltpu.VMEM((8, 128), i32),
            pltpu.VMEM((128, 16), f32),
        ],
    )
    def _kseg(y16_hbm, src2d_hbm, dst2d_hbm, z16_hbm, out_hbm,
              ytabS, accS, sbuf, dbuf, gbuf):
        cid = lax.axis_index("c")
        sid = lax.axis_index("s")
        wid = sid * 2 + cid
        rs = pl.multiple_of(sid * NPS, 128)
        pltpu.sync_copy(y16_hbm.at[pl.ds(rs, NPS)], ytabS.at[pl.ds(rs, NPS)])
        pltpu.sync_copy(z16_hbm.at[pl.ds(rs, NPS)], accS.at[pl.ds(rs, NPS)])
        plsc.subcore_barrier()
        for ch in range(NCH):
            r0 = pl.multiple_of(wid * (EPT // 128) + ch * (CH // 128), 8)
            pltpu.sync_copy(src2d_hbm.at[pl.ds(r0, 8)], sbuf)
            pltpu.sync_copy(dst2d_hbm.at[pl.ds(r0, 8)], dbuf)
            for r in range(8):
                pltpu.sync_copy(ytabS.at[sbuf.at[r]], gbuf)
                pltpu.sync_copy(gbuf, accS.at[dbuf.at[r]], add=True)
        plsc.subcore_barrier()
        pltpu.sync_copy(accS.at[pl.ds(rs, NPS)], out_hbm.at[cid, pl.ds(rs, NPS)])

    return _kseg


def _kseg_sum(y16, src2d, dst2d, z16):
    return _build_kseg()(y16, src2d, dst2d, z16)


@functools.lru_cache(maxsize=None)
def _build_k7():
    @functools.partial(
        pl.kernel,
        out_type=[
            jax.ShapeDtypeStruct((2, NP, HID), f32),
            jax.ShapeDtypeStruct((2, NP, 16), f32),
        ],
        mesh=_mesh(),
        compiler_params=pltpu.CompilerParams(needs_layout_passes=False, use_tc_tiling_on_sc=False),
        scratch_types=[
            pltpu.VMEM_SHARED((NP, HID), f32),
            pltpu.VMEM_SHARED((NP, 16), f32),
            pltpu.VMEM((NP * 4,), f32),
            pltpu.VMEM((8, 128), i32),
            pltpu.VMEM((8, 128), i32),
            pltpu.VMEM((CH,), f32),
            pltpu.VMEM((128, 16), f32),
            pltpu.VMEM((128, HID), f32),
            pltpu.VMEM((16,), f32),
        ],
    )
    def _k7(src2d_hbm, dst2d_hbm, ptab_hbm, xt_hbm, m16_hbm, z64_hbm, z16_hbm,
            nump_hbm, denp_hbm,
            numS, denS, ptab, sbuf, dbuf, exbuf, dpay, grow, mv):
        cid = lax.axis_index("c")
        sid = lax.axis_index("s")
        wid = sid * 2 + cid
        rs = pl.multiple_of(sid * NPS, 128)
        pltpu.sync_copy(ptab_hbm, ptab)
        pltpu.sync_copy(m16_hbm, mv)
        pltpu.sync_copy(z64_hbm.at[pl.ds(rs, NPS)], numS.at[pl.ds(rs, NPS)])
        pltpu.sync_copy(z16_hbm.at[pl.ds(rs, NPS)], denS.at[pl.ds(rs, NPS)])
        pltpu.sync_copy(z16_hbm.at[pl.ds(0, 128)], dpay)
        plsc.subcore_barrier()
        M = mv[...]
        zl = jnp.zeros((16,), i32)
        il = lax.iota(i32, 16)

        def chbody(ch, carry):
            r0 = pl.multiple_of(wid * (EPT // 128) + ch * (CH // 128), 8)
            pltpu.sync_copy(src2d_hbm.at[pl.ds(r0, 8)], sbuf)
            pltpu.sync_copy(dst2d_hbm.at[pl.ds(r0, 8)], dbuf)
            for r in range(8):

                def exbody(j2, carry2, r=r):
                    sv = sbuf[r, pl.ds(j2 * 16, 16)]
                    dv = dbuf[r, pl.ds(j2 * 16, 16)]
                    sv4 = sv * 4
                    dv4 = dv * 4
                    a_s = plsc.load_gather(ptab, [sv4])
                    a_d = plsc.load_gather(ptab, [dv4 + 1])
                    m_s = plsc.load_gather(ptab, [sv4 + 3])
                    m_d = plsc.load_gather(ptab, [dv4 + 3])
                    t = a_s + a_d
                    lg = jnp.where(t >= 0, t, 0.2 * t)
                    ex = jnp.exp(lg - M)
                    exm = jnp.where(m_s * m_d > 0.5, ex, jnp.zeros((16,), f32))
                    exbuf[pl.ds(r * 128 + j2 * 16, 16)] = exm
                    return carry2

                lax.fori_loop(0, 8, exbody, 0)
            for r in range(8):
                pltpu.sync_copy(xt_hbm.at[sbuf.at[r]], grow)
                for jj in range(8):
                    v = exbuf[pl.ds(r * 128 + jj * 16, 16)]
                    plsc.store_scatter(dpay, [jj * 16 + il, zl], v)
                pltpu.sync_copy(dpay, denS.at[dbuf.at[r]], add=True)

                def scbody(g, carry2, r=r):
                    ev = exbuf[pl.ds(r * 128 + g * 16, 16)]
                    for lane in range(16):
                        i = g * 16 + lane
                        e = ev[lane]
                        for q in range(4):
                            grow[i, pl.ds(q * 16, 16)] = e * grow[i, pl.ds(q * 16, 16)]
                    return carry2

                lax.fori_loop(0, 8, scbody, 0)
                pltpu.sync_copy(grow, numS.at[dbuf.at[r]], add=True)
            return carry

        lax.fori_loop(0, NCH, chbody, 0)
        plsc.subcore_barrier()
        pltpu.sync_copy(numS.at[pl.ds(rs, NPS)], nump_hbm.at[cid, pl.ds(rs, NPS)])
        pltpu.sync_copy(denS.at[pl.ds(rs, NPS)], denp_hbm.at[cid, pl.ds(rs, NPS)])

    return _k7


def _k7_gat(src2d, dst2d, ptab, xt, m16, z64, z16):
    return _build_k7()(src2d, dst2d, ptab.reshape(NP * 4), xt, m16, z64, z16)


def _topk_thresholds(skey2d, k):
    t0 = jnp.where(jnp.sum((skey2d >= 0).astype(i32)) >= k, i32(0), i32(-2**31))

    def bit_body(bi, t):
        b = 30 - bi
        cand = t + (i32(1) << b)
        cnt = jnp.sum((skey2d >= cand).astype(i32))
        return jnp.where(cnt >= k, cand, t)

    t = lax.fori_loop(0, 31, bit_body, t0)
    gt = skey2d > t
    eq = skey2d == t
    need = k - jnp.sum(gt.astype(i32))
    pos = (lax.broadcasted_iota(i32, (80, 128), 0) * 128
           + lax.broadcasted_iota(i32, (80, 128), 1))

    def idx_body(bi, t2):
        b = 13 - bi
        cand = t2 | (i32(1) << b)
        cl = jnp.sum((eq & (pos < cand)).astype(i32))
        return jnp.where(cl < need, cand, t2)

    t2 = lax.fori_loop(0, 14, idx_body, i32(0))
    return t, t2


def _skey(score2d):
    bits = lax.bitcast_convert_type(score2d, i32)
    return jnp.where(bits >= 0, bits, bits ^ i32(0x7FFFFFFF))


def _k4_body(aggp_ref, cntp_ref, x_ref, pk_ref, x1_ref, y16_ref):
    cnt = cntp_ref[0, :, 0:1] + cntp_ref[1, :, 0:1]
    agg = (aggp_ref[0] + aggp_ref[1]) / jnp.maximum(cnt, 1.0)
    root = pk_ref[1:5]
    x1 = agg + jnp.dot(x_ref[...], root, preferred_element_type=f32) + pk_ref[0][None, :]
    x1_ref[...] = x1
    y1 = jnp.dot(x1, pk_ref[5][:, None], preferred_element_type=f32)
    y16_ref[...] = jnp.broadcast_to(y1, (NP, 16))


def _k4(aggp, cntp, xp4, pk):
    return pl.pallas_call(
        _k4_body,
        out_shape=[
            jax.ShapeDtypeStruct((NP, HID), f32),
            jax.ShapeDtypeStruct((NP, 16), f32),
        ],
    )(aggp, cntp, xp4, pk)


def _k6_body(x1_ref, nbsp_ref, pk_ref, gw_ref, xt_ref, ptab_ref, m_ref):
    x1 = x1_ref[...]
    nbs = nbsp_ref[0, :, 0:1] + nbsp_ref[1, :, 0:1]
    score = nbs + pk_ref[5, 0] + jnp.dot(x1, pk_ref[4][:, None], preferred_element_type=f32)
    score2d = score.reshape(80, 128)
    pos = (lax.broadcasted_iota(i32, (80, 128), 0) * 128
           + lax.broadcasted_iota(i32, (80, 128), 1))
    sk = jnp.where(pos < N, _skey(score2d), i32(-2**31))
    t, t2 = _topk_thresholds(sk, K1N)
    posc = lax.broadcasted_iota(i32, (NP, 1), 0)
    skc = jnp.where(posc < N, _skey(score), i32(-2**31))
    m1c = (skc > t) | ((skc == t) & (posc <= t2))
    xp = jnp.maximum(x1 * jnp.tanh(score), 0.0)
    xp = xp * (_INV * pk_ref[2][None, :]) + pk_ref[3][None, :]
    xt = jnp.dot(xp, gw_ref[...], preferred_element_type=f32)
    xt_ref[...] = xt
    asrc = jnp.dot(xt, pk_ref[0][:, None], preferred_element_type=f32)
    adst = jnp.dot(xt, pk_ref[1][:, None], preferred_element_type=f32)
    rowm = posc < N
    Ma = jnp.max(jnp.where(rowm, asrc, -3e38))
    Mb = jnp.max(jnp.where(rowm, adst, -3e38))
    M = _lrelu(Ma + Mb, 0.2)
    ssum = asrc + adst
    lg_s = jnp.where(ssum >= 0, ssum, 0.2 * ssum)
    exs = jnp.exp(lg_s - M)
    ptab_ref[...] = jnp.concatenate([asrc, adst, exs, m1c.astype(f32)], axis=1)
    m_ref[...] = jnp.full((8, 16), M, f32)


def _k6(x1, nbsp, pk, gw):
    return pl.pallas_call(
        _k6_body,
        out_shape=[
            jax.ShapeDtypeStruct((NP, HID), f32),
            jax.ShapeDtypeStruct((NP, 4), f32),
            jax.ShapeDtypeStruct((8, 16), f32),
        ],
    )(x1, nbsp, pk, gw)


def _k8_body(nump_ref, denp_ref, xt_ref, ptab_ref, pk_ref, x2_ref, y16_ref):
    xt = xt_ref[...]
    exs = ptab_ref[:, 2:3]
    m1 = ptab_ref[:, 3:4] > 0.5
    den = denp_ref[0, :, 0:1] + denp_ref[1, :, 0:1] + exs
    num = nump_ref[0] + nump_ref[1] + exs * xt
    x2 = num / den + pk_ref[0][None, :]
    x2 = jnp.where(m1, x2, 0.0)
    x2_ref[...] = x2
    y2 = jnp.dot(x2, pk_ref[1][:, None], preferred_element_type=f32)
    y2m = jnp.where(m1, y2, 0.0)
    y16_ref[...] = jnp.broadcast_to(y2m, (NP, 16))


def _k8(nump, denp, xt, ptab, pk):
    return pl.pallas_call(
        _k8_body,
        out_shape=[
            jax.ShapeDtypeStruct((NP, HID), f32),
            jax.ShapeDtypeStruct((NP, 16), f32),
        ],
    )(nump, denp, xt, ptab, pk)


def _k10_body(x2_ref, nb2p_ref, ptab_ref, pk_ref, wih_ref, whh_ref, sb_ref,
              fcw_ref, fcb_ref, out_ref):
    x2 = x2_ref[...]
    m1 = ptab_ref[:, 3:4] > 0.5
    nb2 = nb2p_ref[0, :, 0:1] + nb2p_ref[1, :, 0:1]
    score = nb2 + pk_ref[3, 0] + jnp.dot(x2, pk_ref[2][:, None], preferred_element_type=f32)
    score2d = score.reshape(80, 128)
    posc = lax.broadcasted_iota(i32, (NP, 1), 0)
    skc = jnp.where((posc < N) & m1, _skey(score), i32(-2**31))
    sk = skc.reshape(80, 128)
    t, t2 = _topk_thresholds(sk, K2N)
    m2 = (skc > t) | ((skc == t) & (posc <= t2))
    x3 = jnp.maximum(x2 * jnp.tanh(score), 0.0)
    x3 = x3 * (_INV * pk_ref[0][None, :]) + pk_ref[1][None, :]

    q_star = jnp.zeros((1, 2 * HID), f32)
    hC = jnp.zeros((1, HID), f32)
    cC = jnp.zeros((1, HID), f32)
    for _ in range(5):
        gates = (jnp.dot(q_star, wih_ref[...], preferred_element_type=f32)
                 + sb_ref[0][None, :]
                 + jnp.dot(hC, whh_ref[...], preferred_element_type=f32)
                 + sb_ref[1][None, :])
        ig = jax.nn.sigmoid(gates[:, 0:HID])
        fg = jax.nn.sigmoid(gates[:, HID:2 * HID])
        gg = jnp.tanh(gates[:, 2 * HID:3 * HID])
        og = jax.nn.sigmoid(gates[:, 3 * HID:4 * HID])
        cC = fg * cC + ig * gg
        hC = og * jnp.tanh(cC)
        eatt = jnp.sum(x3 * hC, axis=1, keepdims=True)
        eatt = jnp.where(m2, eatt, -3e38)
        mx = jnp.max(eatt)
        ex = jnp.where(m2, jnp.exp(eatt - mx), 0.0)
        aw = ex / jnp.sum(ex)
        r = jnp.sum(aw * x3, axis=0, keepdims=True)
        q_star = jnp.concatenate([hC, r], axis=1)

    res = jnp.dot(q_star, fcw_ref[...], preferred_element_type=f32) + fcb_ref[0][None, :]
    out_ref[...] = jnp.zeros((8, 128), f32)
    out_ref[0:1, 0:32] = res


def _k10(x2, nb2p, ptab, pk, wih, whh, sb, fcw, fcb):
    return pl.pallas_call(
        _k10_body,
        out_shape=jax.ShapeDtypeStruct((8, 128), f32),
    )(x2, nb2p, ptab, pk, wih, whh, sb, fcw, fcb)


def kernel(x, edge_attr, params, edge_index, batch):
    p = params
    src = edge_index[0]
    dst = edge_index[1]
    srcp = jnp.concatenate([src, jnp.zeros((EP - E,), i32)])
    dstp = jnp.concatenate([dst, jnp.full((EP - E,), NP - 1, i32)])
    src2d = srcp.reshape(EP // 128, 128)
    dst2d = dstp.reshape(EP // 128, 128)
    a3 = jnp.pad(edge_attr[:, 0], (0, EP - E)).reshape(NBE, 16, 128)
    xp4 = jnp.pad(x, ((0, NP - N), (0, 0)))
    z64 = jnp.zeros((NP, HID), f32)
    z16 = jnp.zeros((NP, 16), f32)
    ones16 = jnp.ones((128, 16), f32)

    alpha = p['ea_et_W'][0] @ p['ea_st_W'][:, 0]
    beta = p['ea_et_b'] @ p['ea_st_W'][:, 0] + p['ea_st_b'][0]
    A = p['em_W1'][0] * _INV * p['em_bn_g']
    C = (p['em_b1'] * _INV) * p['em_bn_g'] + p['em_bn_b']
    B2 = p['em_b2'].reshape(NF, HID)
    pk2 = jnp.zeros((8, HID), f32)
    pk2 = pk2.at[0].set(A).at[1].set(C).at[2:6].set(B2)
    pk2 = pk2.at[6, 0].set(alpha).at[6, 1].set(beta)

    pk4 = jnp.zeros((8, HID), f32)
    pk4 = pk4.at[0].set(p['c1_b']).at[1:5].set(p['c1_root']).at[5].set(p['p1_rel_W'][:, 0])

    pk6 = jnp.zeros((8, HID), f32)
    pk6 = (pk6.at[0].set(p['g_asrc']).at[1].set(p['g_adst'])
              .at[2].set(p['bn1_g']).at[3].set(p['bn1_b'])
              .at[4].set(p['p1_root_W'][:, 0]).at[5, 0].set(p['p1_rel_b'][0]))

    pk8 = jnp.zeros((8, HID), f32)
    pk8 = pk8.at[0].set(p['g_b']).at[1].set(p['p2_rel_W'][:, 0])

    pk10 = jnp.zeros((8, HID), f32)
    pk10 = (pk10.at[0].set(p['bn2_g']).at[1].set(p['bn2_b'])
                .at[2].set(p['p2_root_W'][:, 0]).at[3, 0].set(p['p2_rel_b'][0]))
    wih = p['s2s_Wih'].T
    whh = p['s2s_Whh'].T
    sb = jnp.zeros((8, 4 * HID), f32)
    sb = sb.at[0].set(p['s2s_bih']).at[1].set(p['s2s_bhh'])
    fcw = p['fc_W']
    fcb = jnp.zeros((8, 32), f32).at[0].set(p['fc_b'])

    xs = _k1_gather(xp4.reshape(NP * 4), srcp)
    msg = _k2_msg(a3, xs, pk2, p['em_W2'])
    aggp, cntp = _k3_scatter(msg, dst2d, z64, z16, ones16)
    x1, y16 = _k4(aggp, cntp, xp4, pk4)
    nbsp = _kseg_sum(y16, src2d, dst2d, z16)
    xt, ptab, m8 = _k6(x1, nbsp, pk6, p['g_W'])
    m16 = m8.reshape(128)[0:16]
    nump, denp = _k7_gat(src2d, dst2d, ptab, xt, m16, z64, z16)
    x2, y216 = _k8(nump, denp, xt, ptab, pk8)
    nb2p = _kseg_sum(y216, src2d, dst2d, z16)
    out = _k10(x2, nb2p, ptab, pk10, wih, whh, sb, fcw, fcb)
    return out[0:1, 0:32]


# speedup vs baseline: 10.4681x; 1.1175x over previous
import functools
import math

import jax
import jax.numpy as jnp
from jax import lax
from jax.experimental import pallas as pl
from jax.experimental.pallas import tpu as pltpu
from jax.experimental.pallas import tpu_sc as plsc

N = 10000
NP = 10240
E = 160000
EP = 163840
HID = 64
NF = 4
K1N = (N + 1) // 2
K2N = (K1N + 1) // 2
NW = 32
EPT = EP // NW
CH = 1024
NCH = EPT // CH
BE = 2048
NBE = EP // BE
NPS = NP // 16
_INV = 1.0 / math.sqrt(1.0 + 1e-5)

f32 = jnp.float32
i32 = jnp.int32


def _mesh():
    return plsc.VectorSubcoreMesh(core_axis_name="c", subcore_axis_name="s")


def _lrelu(x, s=0.01):
    return jnp.where(x >= 0, x, s * x)


@functools.lru_cache(maxsize=None)
def _build_k1():
    @functools.partial(
        pl.kernel,
        out_type=jax.ShapeDtypeStruct((8, EP), f32),
        mesh=_mesh(),
        compiler_params=pltpu.CompilerParams(needs_layout_passes=False, use_tc_tiling_on_sc=False),
        scratch_types=[
            pltpu.VMEM((NP * 4,), f32),
            pltpu.VMEM((EPT,), i32),
            pltpu.VMEM((4, EPT), f32),
        ],
    )
    def _k1(x_hbm, src_hbm, xs_hbm, xtab, sidx, xsb):
        cid = lax.axis_index("c")
        sid = lax.axis_index("s")
        wid = sid * 2 + cid
        base = pl.multiple_of(wid * EPT, 128)
        pltpu.sync_copy(x_hbm, xtab)
        pltpu.sync_copy(src_hbm.at[pl.ds(base, EPT)], sidx)

        def body(j, carry):
            iv = sidx[pl.ds(j * 16, 16)]
            iv4 = iv * 4
            for f in range(4):
                g = plsc.load_gather(xtab, [iv4 + f])
                xsb[f, pl.ds(j * 16, 16)] = g
            return carry

        lax.fori_loop(0, EPT // 16, body, 0)
        for f in range(4):
            pltpu.sync_copy(xsb.at[f], xs_hbm.at[f, pl.ds(base, EPT)])

    return _k1


def _k1_gather(xp4, srcp):
    return _build_k1()(xp4, srcp)


def _k2a_body(a_ref, pk_ref, stat_ref):
    a = a_ref[...]
    alpha = pk_ref[6, 0]
    beta = pk_ref[6, 1]
    s_l = _lrelu(alpha * a + beta)
    pos = (lax.broadcasted_iota(i32, (EP // 128, 128), 0) * 128
           + lax.broadcasted_iota(i32, (EP // 128, 128), 1))
    valid = pos < E
    mb = jnp.max(jnp.where(valid, s_l, -3e38))
    sb = jnp.sum(jnp.where(valid, jnp.exp(s_l - mb), 0.0))
    rr = lax.broadcasted_iota(i32, (8, 128), 0)
    cc = lax.broadcasted_iota(i32, (8, 128), 1)
    stat_ref[...] = (jnp.where((rr == 0) & (cc == 0), mb, 0.0)
                     + jnp.where((rr == 0) & (cc == 1), sb, 0.0))


def _k2a_stats(a2d, pk):
    return pl.pallas_call(
        _k2a_body,
        out_shape=jax.ShapeDtypeStruct((8, 128), f32),
    )(a2d, pk)


BE2 = 4096
NBE2 = EP // BE2


def _k2_body(a_ref, xs_ref, pk_ref, w2_ref, stat_ref, msg_ref):
    i = pl.program_id(0)
    del i
    ab = a_ref[0].reshape(BE2)
    alpha = pk_ref[6, 0]
    beta = pk_ref[6, 1]
    s_l = _lrelu(alpha * ab + beta)
    m0 = stat_ref[0, 0]
    s0 = stat_ref[0, 1]
    sm = jnp.exp(s_l - m0) / s0
    ea = _lrelu(ab * sm)
    A = pk_ref[0]
    C = pk_ref[1]
    h = _lrelu(ea[:, None] * A[None, :] + C[None, :])
    acc = jnp.zeros((BE2, HID), f32)
    for fdim in range(4):
        wf = w2_ref[:, fdim * HID:(fdim + 1) * HID]
        hf = jnp.dot(h, wf, preferred_element_type=f32) + pk_ref[2 + fdim][None, :]
        acc = acc + xs_ref[fdim][:, None] * hf
    msg_ref[...] = acc


def _k2_msg(a3, xs, pk, w2, stats):
    return pl.pallas_call(
        _k2_body,
        grid=(NBE2,),
        in_specs=[
            pl.BlockSpec((1, 32, 128), lambda i: (i, 0, 0)),
            pl.BlockSpec((8, BE2), lambda i: (0, i)),
            pl.BlockSpec((8, HID), lambda i: (0, 0)),
            pl.BlockSpec((HID, NF * HID), lambda i: (0, 0)),
            pl.BlockSpec((8, 128), lambda i: (0, 0)),
        ],
        out_specs=pl.BlockSpec((BE2, HID), lambda i: (i, 0)),
        out_shape=jax.ShapeDtypeStruct((EP, HID), f32),
    )(a3, xs, pk, w2, stats)


@functools.lru_cache(maxsize=None)
def _build_k3():
    @functools.partial(
        pl.kernel,
        out_type=[
            jax.ShapeDtypeStruct((2, NP, HID), f32),
            jax.ShapeDtypeStruct((2, NP, 16), f32),
        ],
        mesh=_mesh(),
        compiler_params=pltpu.CompilerParams(needs_layout_passes=False, use_tc_tiling_on_sc=False),
        scratch_types=[
            pltpu.VMEM_SHARED((NP, HID), f32),
            pltpu.VMEM_SHARED((NP, 16), f32),
            pltpu.VMEM((8, 128), i32),
            pltpu.VMEM((CH, HID), f32),
            pltpu.VMEM((128, 16), f32),
        ],
    )
    def _k3(msg_hbm, dst2d_hbm, z64_hbm, z16_hbm, ones_hbm,
            aggp_hbm, cntp_hbm, aggS, cntS, dbuf, mbuf, onesv):
        cid = lax.axis_index("c")
        sid = lax.axis_index("s")
        wid = sid * 2 + cid
        rs = pl.multiple_of(sid * NPS, 128)
        pltpu.sync_copy(z64_hbm.at[pl.ds(rs, NPS)], aggS.at[pl.ds(rs, NPS)])
        pltpu.sync_copy(z16_hbm.at[pl.ds(rs, NPS)], cntS.at[pl.ds(rs, NPS)])
        pltpu.sync_copy(ones_hbm, onesv)
        plsc.subcore_barrier()
        for ch in range(NCH):
            e0 = pl.multiple_of(wid * EPT + ch * CH, 128)
            r0 = pl.multiple_of(wid * (EPT // 128) + ch * (CH // 128), 8)
            pltpu.sync_copy(dst2d_hbm.at[pl.ds(r0, 8)], dbuf)
            pltpu.sync_copy(msg_hbm.at[pl.ds(e0, CH)], mbuf)
            for r in range(8):
                pltpu.sync_copy(mbuf.at[pl.ds(r * 128, 128)],
                                aggS.at[dbuf.at[r]], add=True)
                pltpu.sync_copy(onesv, cntS.at[dbuf.at[r]], add=True)
        plsc.subcore_barrier()
        pltpu.sync_copy(aggS.at[pl.ds(rs, NPS)], aggp_hbm.at[cid, pl.ds(rs, NPS)])
        pltpu.sync_copy(cntS.at[pl.ds(rs, NPS)], cntp_hbm.at[cid, pl.ds(rs, NPS)])

    return _k3


def _k3_scatter(msg, dst2d, z64, z16, ones16):
    return _build_k3()(msg, dst2d, z64, z16, ones16)


@functools.lru_cache(maxsize=None)
def _build_kseg():
    @functools.partial(
        pl.kernel,
        out_type=jax.ShapeDtypeStruct((2, NP, 16), f32),
        mesh=_mesh(),
        compiler_params=pltpu.CompilerParams(needs_layout_passes=False, use_tc_tiling_on_sc=False),
        scratch_types=[
            pltpu.VMEM_SHARED((NP, 16), f32),
            pltpu.VMEM_SHARED((NP, 16), f32),
            pltpu.VMEM((8, 128), i32),
            pltpu.VMEM((8, 128), i32),
            pltpu.VMEM((128, 16), f32),
        ],
    )
    def _kseg(y16_hbm, src2d_hbm, dst2d_hbm, z16_hbm, out_hbm,
              ytabS, accS, sbuf, dbuf, gbuf):
        cid = lax.axis_index("c")
        sid = lax.axis_index("s")
        wid = sid * 2 + cid
        rs = pl.multiple_of(sid * NPS, 128)
        pltpu.sync_copy(y16_hbm.at[pl.ds(rs, NPS)], ytabS.at[pl.ds(rs, NPS)])
        pltpu.sync_copy(z16_hbm.at[pl.ds(rs, NPS)], accS.at[pl.ds(rs, NPS)])
        plsc.subcore_barrier()
        for ch in range(NCH):
            r0 = pl.multiple_of(wid * (EPT // 128) + ch * (CH // 128), 8)
            pltpu.sync_copy(src2d_hbm.at[pl.ds(r0, 8)], sbuf)
            pltpu.sync_copy(dst2d_hbm.at[pl.ds(r0, 8)], dbuf)
            for r in range(8):
                pltpu.sync_copy(ytabS.at[sbuf.at[r]], gbuf)
                pltpu.sync_copy(gbuf, accS.at[dbuf.at[r]], add=True)
        plsc.subcore_barrier()
        pltpu.sync_copy(accS.at[pl.ds(rs, NPS)], out_hbm.at[cid, pl.ds(rs, NPS)])

    return _kseg


def _kseg_sum(y16, src2d, dst2d, z16):
    return _build_kseg()(y16, src2d, dst2d, z16)


@functools.lru_cache(maxsize=None)
def _build_k7():
    @functools.partial(
        pl.kernel,
        out_type=[
            jax.ShapeDtypeStruct((2, NP, HID), f32),
            jax.ShapeDtypeStruct((2, NP, 16), f32),
        ],
        mesh=_mesh(),
        compiler_params=pltpu.CompilerParams(needs_layout_passes=False, use_tc_tiling_on_sc=False),
        scratch_types=[
            pltpu.VMEM_SHARED((NP, HID), f32),
            pltpu.VMEM_SHARED((NP, 16), f32),
            pltpu.VMEM((NP * 4,), f32),
            pltpu.VMEM((8, 128), i32),
            pltpu.VMEM((8, 128), i32),
            pltpu.VMEM((CH,), f32),
            pltpu.VMEM((128, 16), f32),
            pltpu.VMEM((128, HID), f32),
            pltpu.VMEM((16,), f32),
        ],
    )
    def _k7(src2d_hbm, dst2d_hbm, ptab_hbm, xt_hbm, m16_hbm, z64_hbm, z16_hbm,
            nump_hbm, denp_hbm,
            numS, denS, ptab, sbuf, dbuf, exbuf, dpay, grow, mv):
        cid = lax.axis_index("c")
        sid = lax.axis_index("s")
        wid = sid * 2 + cid
        rs = pl.multiple_of(sid * NPS, 128)
        pltpu.sync_copy(ptab_hbm, ptab)
        pltpu.sync_copy(m16_hbm, mv)
        pltpu.sync_copy(z64_hbm.at[pl.ds(rs, NPS)], numS.at[pl.ds(rs, NPS)])
        pltpu.sync_copy(z16_hbm.at[pl.ds(rs, NPS)], denS.at[pl.ds(rs, NPS)])
        pltpu.sync_copy(z16_hbm.at[pl.ds(0, 128)], dpay)
        plsc.subcore_barrier()
        M = mv[...]
        zl = jnp.zeros((16,), i32)
        il = lax.iota(i32, 16)

        def chbody(ch, carry):
            r0 = pl.multiple_of(wid * (EPT // 128) + ch * (CH // 128), 8)
            pltpu.sync_copy(src2d_hbm.at[pl.ds(r0, 8)], sbuf)
            pltpu.sync_copy(dst2d_hbm.at[pl.ds(r0, 8)], dbuf)
            for r in range(8):

                def exbody(j2, carry2, r=r):
                    sv = sbuf[r, pl.ds(j2 * 16, 16)]
                    dv = dbuf[r, pl.ds(j2 * 16, 16)]
                    sv4 = sv * 4
                    dv4 = dv * 4
                    a_s = plsc.load_gather(ptab, [sv4])
                    a_d = plsc.load_gather(ptab, [dv4 + 1])
                    m_s = plsc.load_gather(ptab, [sv4 + 3])
                    m_d = plsc.load_gather(ptab, [dv4 + 3])
                    t = a_s + a_d
                    lg = jnp.where(t >= 0, t, 0.2 * t)
                    ex = jnp.exp(lg - M)
                    exm = jnp.where(m_s * m_d > 0.5, ex, jnp.zeros((16,), f32))
                    exbuf[pl.ds(r * 128 + j2 * 16, 16)] = exm
                    return carry2

                lax.fori_loop(0, 8, exbody, 0)
            for r in range(8):
                pltpu.sync_copy(xt_hbm.at[sbuf.at[r]], grow)
                for jj in range(8):
                    v = exbuf[pl.ds(r * 128 + jj * 16, 16)]
                    plsc.store_scatter(dpay, [jj * 16 + il, zl], v)
                pltpu.sync_copy(dpay, denS.at[dbuf.at[r]], add=True)

                def scbody(g, carry2, r=r):
                    ev = exbuf[pl.ds(r * 128 + g * 16, 16)]
                    for lane in range(16):
                        i = g * 16 + lane
                        e = ev[lane]
                        for q in range(4):
                            grow[i, pl.ds(q * 16, 16)] = e * grow[i, pl.ds(q * 16, 16)]
                    return carry2

                lax.fori_loop(0, 8, scbody, 0)
                pltpu.sync_copy(grow, numS.at[dbuf.at[r]], add=True)
            return carry

        lax.fori_loop(0, NCH, chbody, 0)
        plsc.subcore_barrier()
        pltpu.sync_copy(numS.at[pl.ds(rs, NPS)], nump_hbm.at[cid, pl.ds(rs, NPS)])
        pltpu.sync_copy(denS.at[pl.ds(rs, NPS)], denp_hbm.at[cid, pl.ds(rs, NPS)])

    return _k7


def _k7_gat(src2d, dst2d, ptab, xt, m16, z64, z16):
    return _build_k7()(src2d, dst2d, ptab.reshape(NP * 4), xt, m16, z64, z16)


def _topk_thresholds(skey2d, k):
    t0 = jnp.where(jnp.sum((skey2d >= 0).astype(i32)) >= k, i32(0), i32(-2**31))

    def bit_body(bi, t):
        b = 30 - bi
        cand = t + (i32(1) << b)
        cnt = jnp.sum((skey2d >= cand).astype(i32))
        return jnp.where(cnt >= k, cand, t)

    t = lax.fori_loop(0, 31, bit_body, t0)
    gt = skey2d > t
    eq = skey2d == t
    need = k - jnp.sum(gt.astype(i32))
    pos = (lax.broadcasted_iota(i32, (80, 128), 0) * 128
           + lax.broadcasted_iota(i32, (80, 128), 1))

    def idx_body(bi, t2):
        b = 13 - bi
        cand = t2 | (i32(1) << b)
        cl = jnp.sum((eq & (pos < cand)).astype(i32))
        return jnp.where(cl < need, cand, t2)

    t2 = lax.fori_loop(0, 14, idx_body, i32(0))
    return t, t2


def _skey(score2d):
    bits = lax.bitcast_convert_type(score2d, i32)
    return jnp.where(bits >= 0, bits, bits ^ i32(0x7FFFFFFF))


def _k4_body(aggp_ref, cntp_ref, x_ref, pk_ref, x1_ref, y16_ref):
    cnt = cntp_ref[0, :, 0:1] + cntp_ref[1, :, 0:1]
    agg = (aggp_ref[0] + aggp_ref[1]) / jnp.maximum(cnt, 1.0)
    root = pk_ref[1:5]
    x1 = agg + jnp.dot(x_ref[...], root, preferred_element_type=f32) + pk_ref[0][None, :]
    x1_ref[...] = x1
    y1 = jnp.dot(x1, pk_ref[5][:, None], preferred_element_type=f32)
    y16_ref[...] = jnp.broadcast_to(y1, (NP, 16))


def _k4(aggp, cntp, xp4, pk):
    return pl.pallas_call(
        _k4_body,
        out_shape=[
            jax.ShapeDtypeStruct((NP, HID), f32),
            jax.ShapeDtypeStruct((NP, 16), f32),
        ],
    )(aggp, cntp, xp4, pk)


def _k6_body(x1_ref, nbsp_ref, pk_ref, gw_ref, xt_ref, ptab_ref, m_ref):
    x1 = x1_ref[...]
    nbs = nbsp_ref[0, :, 0:1] + nbsp_ref[1, :, 0:1]
    score = nbs + pk_ref[5, 0] + jnp.dot(x1, pk_ref[4][:, None], preferred_element_type=f32)
    score2d = score.reshape(80, 128)
    pos = (lax.broadcasted_iota(i32, (80, 128), 0) * 128
           + lax.broadcasted_iota(i32, (80, 128), 1))
    sk = jnp.where(pos < N, _skey(score2d), i32(-2**31))
    t, t2 = _topk_thresholds(sk, K1N)
    posc = lax.broadcasted_iota(i32, (NP, 1), 0)
    skc = jnp.where(posc < N, _skey(score), i32(-2**31))
    m1c = (skc > t) | ((skc == t) & (posc <= t2))
    xp = jnp.maximum(x1 * jnp.tanh(score), 0.0)
    xp = xp * (_INV * pk_ref[2][None, :]) + pk_ref[3][None, :]
    xt = jnp.dot(xp, gw_ref[...], preferred_element_type=f32)
    xt_ref[...] = xt
    asrc = jnp.dot(xt, pk_ref[0][:, None], preferred_element_type=f32)
    adst = jnp.dot(xt, pk_ref[1][:, None], preferred_element_type=f32)
    rowm = posc < N
    Ma = jnp.max(jnp.where(rowm, asrc, -3e38))
    Mb = jnp.max(jnp.where(rowm, adst, -3e38))
    M = _lrelu(Ma + Mb, 0.2)
    ssum = asrc + adst
    lg_s = jnp.where(ssum >= 0, ssum, 0.2 * ssum)
    exs = jnp.exp(lg_s - M)
    ptab_ref[...] = jnp.concatenate([asrc, adst, exs, m1c.astype(f32)], axis=1)
    m_ref[...] = jnp.full((8, 16), M, f32)


def _k6(x1, nbsp, pk, gw):
    return pl.pallas_call(
        _k6_body,
        out_shape=[
            jax.ShapeDtypeStruct((NP, HID), f32),
            jax.ShapeDtypeStruct((NP, 4), f32),
            jax.ShapeDtypeStruct((8, 16), f32),
        ],
    )(x1, nbsp, pk, gw)


def _k8_body(nump_ref, denp_ref, xt_ref, ptab_ref, pk_ref, x2_ref, y16_ref):
    xt = xt_ref[...]
    exs = ptab_ref[:, 2:3]
    m1 = ptab_ref[:, 3:4] > 0.5
    den = denp_ref[0, :, 0:1] + denp_ref[1, :, 0:1] + exs
    num = nump_ref[0] + nump_ref[1] + exs * xt
    x2 = num / den + pk_ref[0][None, :]
    x2 = jnp.where(m1, x2, 0.0)
    x2_ref[...] = x2
    y2 = jnp.dot(x2, pk_ref[1][:, None], preferred_element_type=f32)
    y2m = jnp.where(m1, y2, 0.0)
    y16_ref[...] = jnp.broadcast_to(y2m, (NP, 16))


def _k8(nump, denp, xt, ptab, pk):
    return pl.pallas_call(
        _k8_body,
        out_shape=[
            jax.ShapeDtypeStruct((NP, HID), f32),
            jax.ShapeDtypeStruct((NP, 16), f32),
        ],
    )(nump, denp, xt, ptab, pk)


def _k10_body(x2_ref, nb2p_ref, ptab_ref, pk_ref, wih_ref, whh_ref, sb_ref,
              fcw_ref, fcb_ref, out_ref):
    x2 = x2_ref[...]
    m1 = ptab_ref[:, 3:4] > 0.5
    nb2 = nb2p_ref[0, :, 0:1] + nb2p_ref[1, :, 0:1]
    score = nb2 + pk_ref[3, 0] + jnp.dot(x2, pk_ref[2][:, None], preferred_element_type=f32)
    score2d = score.reshape(80, 128)
    posc = lax.broadcasted_iota(i32, (NP, 1), 0)
    skc = jnp.where((posc < N) & m1, _skey(score), i32(-2**31))
    sk = skc.reshape(80, 128)
    t, t2 = _topk_thresholds(sk, K2N)
    m2 = (skc > t) | ((skc == t) & (posc <= t2))
    x3 = jnp.maximum(x2 * jnp.tanh(score), 0.0)
    x3 = x3 * (_INV * pk_ref[0][None, :]) + pk_ref[1][None, :]

    q_star = jnp.zeros((1, 2 * HID), f32)
    hC = jnp.zeros((1, HID), f32)
    cC = jnp.zeros((1, HID), f32)
    for _ in range(5):
        gates = (jnp.dot(q_star, wih_ref[...], preferred_element_type=f32)
                 + sb_ref[0][None, :]
                 + jnp.dot(hC, whh_ref[...], preferred_element_type=f32)
                 + sb_ref[1][None, :])
        ig = jax.nn.sigmoid(gates[:, 0:HID])
        fg = jax.nn.sigmoid(gates[:, HID:2 * HID])
        gg = jnp.tanh(gates[:, 2 * HID:3 * HID])
        og = jax.nn.sigmoid(gates[:, 3 * HID:4 * HID])
        cC = fg * cC + ig * gg
        hC = og * jnp.tanh(cC)
        eatt = jnp.sum(x3 * hC, axis=1, keepdims=True)
        eatt = jnp.where(m2, eatt, -3e38)
        mx = jnp.max(eatt)
        ex = jnp.where(m2, jnp.exp(eatt - mx), 0.0)
        aw = ex / jnp.sum(ex)
        r = jnp.sum(aw * x3, axis=0, keepdims=True)
        q_star = jnp.concatenate([hC, r], axis=1)

    res = jnp.dot(q_star, fcw_ref[...], preferred_element_type=f32) + fcb_ref[0][None, :]
    out_ref[...] = jnp.zeros((8, 128), f32)
    out_ref[0:1, 0:32] = res


def _k10(x2, nb2p, ptab, pk, wih, whh, sb, fcw, fcb):
    return pl.pallas_call(
        _k10_body,
        out_shape=jax.ShapeDtypeStruct((8, 128), f32),
    )(x2, nb2p, ptab, pk, wih, whh, sb, fcw, fcb)


def kernel(x, edge_attr, params, edge_index, batch):
    p = params
    src = edge_index[0]
    dst = edge_index[1]
    srcp = jnp.concatenate([src, jnp.zeros((EP - E,), i32)])
    dstp = jnp.concatenate([dst, jnp.full((EP - E,), NP - 1, i32)])
    src2d = srcp.reshape(EP // 128, 128)
    dst2d = dstp.reshape(EP // 128, 128)
    apad = jnp.pad(edge_attr[:, 0], (0, EP - E))
    a3 = apad.reshape(NBE2, 32, 128)
    a2d = apad.reshape(EP // 128, 128)
    xp4 = jnp.pad(x, ((0, NP - N), (0, 0)))
    z64 = jnp.zeros((NP, HID), f32)
    z16 = jnp.zeros((NP, 16), f32)
    ones16 = jnp.ones((128, 16), f32)

    alpha = p['ea_et_W'][0] @ p['ea_st_W'][:, 0]
    beta = p['ea_et_b'] @ p['ea_st_W'][:, 0] + p['ea_st_b'][0]
    A = p['em_W1'][0] * _INV * p['em_bn_g']
    C = (p['em_b1'] * _INV) * p['em_bn_g'] + p['em_bn_b']
    B2 = p['em_b2'].reshape(NF, HID)
    pk2 = jnp.zeros((8, HID), f32)
    pk2 = pk2.at[0].set(A).at[1].set(C).at[2:6].set(B2)
    pk2 = pk2.at[6, 0].set(alpha).at[6, 1].set(beta)

    pk4 = jnp.zeros((8, HID), f32)
    pk4 = pk4.at[0].set(p['c1_b']).at[1:5].set(p['c1_root']).at[5].set(p['p1_rel_W'][:, 0])

    pk6 = jnp.zeros((8, HID), f32)
    pk6 = (pk6.at[0].set(p['g_asrc']).at[1].set(p['g_adst'])
              .at[2].set(p['bn1_g']).at[3].set(p['bn1_b'])
              .at[4].set(p['p1_root_W'][:, 0]).at[5, 0].set(p['p1_rel_b'][0]))

    pk8 = jnp.zeros((8, HID), f32)
    pk8 = pk8.at[0].set(p['g_b']).at[1].set(p['p2_rel_W'][:, 0])

    pk10 = jnp.zeros((8, HID), f32)
    pk10 = (pk10.at[0].set(p['bn2_g']).at[1].set(p['bn2_b'])
                .at[2].set(p['p2_root_W'][:, 0]).at[3, 0].set(p['p2_rel_b'][0]))
    wih = p['s2s_Wih'].T
    whh = p['s2s_Whh'].T
    sb = jnp.zeros((8, 4 * HID), f32)
    sb = sb.at[0].set(p['s2s_bih']).at[1].set(p['s2s_bhh'])
    fcw = p['fc_W']
    fcb = jnp.zeros((8, 32), f32).at[0].set(p['fc_b'])

    xs = _k1_gather(xp4.reshape(NP * 4), srcp)
    stats = _k2a_stats(a2d, pk2)
    msg = _k2_msg(a3, xs, pk2, p['em_W2'], stats)
    aggp, cntp = _k3_scatter(msg, dst2d, z64, z16, ones16)
    x1, y16 = _k4(aggp, cntp, xp4, pk4)
    nbsp = _kseg_sum(y16, src2d, dst2d, z16)
    xt, ptab, m8 = _k6(x1, nbsp, pk6, p['g_W'])
    m16 = m8.reshape(128)[0:16]
    nump, denp = _k7_gat(src2d, dst2d, ptab, xt, m16, z64, z16)
    x2, y216 = _k8(nump, denp, xt, ptab, pk8)
    nb2p = _kseg_sum(y216, src2d, dst2d, z16)
    out = _k10(x2, nb2p, ptab, pk10, wih, whh, sb, fcw, fcb)
    return out[0:1, 0:32]


# speedup vs baseline: 11.5131x; 1.0998x over previous
import functools
import math

import jax
import jax.numpy as jnp
from jax import lax
from jax.experimental import pallas as pl
from jax.experimental.pallas import tpu as pltpu
from jax.experimental.pallas import tpu_sc as plsc

N = 10000
NP = 10240
E = 160000
EP = 163840
HID = 64
NF = 4
K1N = (N + 1) // 2
K2N = (K1N + 1) // 2
NW = 32
EPT = EP // NW
CH = 1024
NCH = EPT // CH
BE = 2048
NBE = EP // BE
NPS = NP // 16
_INV = 1.0 / math.sqrt(1.0 + 1e-5)

f32 = jnp.float32
i32 = jnp.int32


def _mesh():
    return plsc.VectorSubcoreMesh(core_axis_name="c", subcore_axis_name="s")


def _lrelu(x, s=0.01):
    return jnp.where(x >= 0, x, s * x)


@functools.lru_cache(maxsize=None)
def _build_k1():
    @functools.partial(
        pl.kernel,
        out_type=jax.ShapeDtypeStruct((8, EP), f32),
        mesh=_mesh(),
        compiler_params=pltpu.CompilerParams(needs_layout_passes=False, use_tc_tiling_on_sc=False),
        scratch_types=[
            pltpu.VMEM((NP * 4,), f32),
            pltpu.VMEM((EPT,), i32),
            pltpu.VMEM((4, EPT), f32),
        ],
    )
    def _k1(x_hbm, src_hbm, xs_hbm, xtab, sidx, xsb):
        cid = lax.axis_index("c")
        sid = lax.axis_index("s")
        wid = sid * 2 + cid
        base = pl.multiple_of(wid * EPT, 128)
        pltpu.sync_copy(x_hbm, xtab)
        pltpu.sync_copy(src_hbm.at[pl.ds(base, EPT)], sidx)

        def body(j, carry):
            iv = sidx[pl.ds(j * 16, 16)]
            iv4 = iv * 4
            for f in range(4):
                g = plsc.load_gather(xtab, [iv4 + f])
                xsb[f, pl.ds(j * 16, 16)] = g
            return carry

        lax.fori_loop(0, EPT // 16, body, 0)
        for f in range(4):
            pltpu.sync_copy(xsb.at[f], xs_hbm.at[f, pl.ds(base, EPT)])

    return _k1


def _k1_gather(xp4, srcp):
    return _build_k1()(xp4, srcp)


def _k2a_body(a_ref, pk_ref, stat_ref):
    a = a_ref[...]
    alpha = pk_ref[6, 0]
    beta = pk_ref[6, 1]
    s_l = _lrelu(alpha * a + beta)
    pos = (lax.broadcasted_iota(i32, (EP // 128, 128), 0) * 128
           + lax.broadcasted_iota(i32, (EP // 128, 128), 1))
    valid = pos < E
    mb = jnp.max(jnp.where(valid, s_l, -3e38))
    sb = jnp.sum(jnp.where(valid, jnp.exp(s_l - mb), 0.0))
    rr = lax.broadcasted_iota(i32, (8, 128), 0)
    cc = lax.broadcasted_iota(i32, (8, 128), 1)
    stat_ref[...] = (jnp.where((rr == 0) & (cc == 0), mb, 0.0)
                     + jnp.where((rr == 0) & (cc == 1), sb, 0.0))


def _k2a_stats(a2d, pk):
    return pl.pallas_call(
        _k2a_body,
        out_shape=jax.ShapeDtypeStruct((8, 128), f32),
    )(a2d, pk)


BE2 = 4096
NBE2 = EP // BE2


def _k2_body(a_ref, xs_ref, pk_ref, w2_ref, stat_ref, msg_ref):
    i = pl.program_id(0)
    del i
    ab = a_ref[0].reshape(BE2)
    alpha = pk_ref[6, 0]
    beta = pk_ref[6, 1]
    s_l = _lrelu(alpha * ab + beta)
    m0 = stat_ref[0, 0]
    s0 = stat_ref[0, 1]
    sm = jnp.exp(s_l - m0) / s0
    ea = _lrelu(ab * sm)
    A = pk_ref[0]
    C = pk_ref[1]
    h = _lrelu(ea[:, None] * A[None, :] + C[None, :])
    acc = jnp.zeros((BE2, HID), f32)
    for fdim in range(4):
        wf = w2_ref[:, fdim * HID:(fdim + 1) * HID]
        hf = jnp.dot(h, wf, preferred_element_type=f32) + pk_ref[2 + fdim][None, :]
        acc = acc + xs_ref[fdim][:, None] * hf
    msg_ref[...] = acc


def _k2_msg(a3, xs, pk, w2, stats):
    return pl.pallas_call(
        _k2_body,
        grid=(NBE2,),
        in_specs=[
            pl.BlockSpec((1, 32, 128), lambda i: (i, 0, 0)),
            pl.BlockSpec((8, BE2), lambda i: (0, i)),
            pl.BlockSpec((8, HID), lambda i: (0, 0)),
            pl.BlockSpec((HID, NF * HID), lambda i: (0, 0)),
            pl.BlockSpec((8, 128), lambda i: (0, 0)),
        ],
        out_specs=pl.BlockSpec((BE2, HID), lambda i: (i, 0)),
        out_shape=jax.ShapeDtypeStruct((EP, HID), f32),
    )(a3, xs, pk, w2, stats)


@functools.lru_cache(maxsize=None)
def _build_k3():
    @functools.partial(
        pl.kernel,
        out_type=[
            jax.ShapeDtypeStruct((2, NP, HID), f32),
            jax.ShapeDtypeStruct((2, NP, 16), f32),
        ],
        mesh=_mesh(),
        compiler_params=pltpu.CompilerParams(needs_layout_passes=False, use_tc_tiling_on_sc=False),
        scratch_types=[
            pltpu.VMEM_SHARED((NP, HID), f32),
            pltpu.VMEM_SHARED((NP, 16), f32),
            pltpu.VMEM((8, 128), i32),
            pltpu.VMEM((CH, HID), f32),
            pltpu.VMEM((128, 16), f32),
        ],
    )
    def _k3(msg_hbm, dst2d_hbm, z64_hbm, z16_hbm, ones_hbm,
            aggp_hbm, cntp_hbm, aggS, cntS, dbuf, mbuf, onesv):
        cid = lax.axis_index("c")
        sid = lax.axis_index("s")
        wid = sid * 2 + cid
        rs = pl.multiple_of(sid * NPS, 128)
        pltpu.sync_copy(z64_hbm.at[pl.ds(rs, NPS)], aggS.at[pl.ds(rs, NPS)])
        pltpu.sync_copy(z16_hbm.at[pl.ds(rs, NPS)], cntS.at[pl.ds(rs, NPS)])
        pltpu.sync_copy(ones_hbm, onesv)
        plsc.subcore_barrier()
        for ch in range(NCH):
            e0 = pl.multiple_of(wid * EPT + ch * CH, 128)
            r0 = pl.multiple_of(wid * (EPT // 128) + ch * (CH // 128), 8)
            pltpu.sync_copy(dst2d_hbm.at[pl.ds(r0, 8)], dbuf)
            pltpu.sync_copy(msg_hbm.at[pl.ds(e0, CH)], mbuf)
            for r in range(8):
                pltpu.sync_copy(mbuf.at[pl.ds(r * 128, 128)],
                                aggS.at[dbuf.at[r]], add=True)
                pltpu.sync_copy(onesv, cntS.at[dbuf.at[r]], add=True)
        plsc.subcore_barrier()
        pltpu.sync_copy(aggS.at[pl.ds(rs, NPS)], aggp_hbm.at[cid, pl.ds(rs, NPS)])
        pltpu.sync_copy(cntS.at[pl.ds(rs, NPS)], cntp_hbm.at[cid, pl.ds(rs, NPS)])

    return _k3


def _k3_scatter(msg, dst2d, z64, z16, ones16):
    return _build_k3()(msg, dst2d, z64, z16, ones16)


@functools.lru_cache(maxsize=None)
def _build_kseg():
    @functools.partial(
        pl.kernel,
        out_type=jax.ShapeDtypeStruct((2, NP, 16), f32),
        mesh=_mesh(),
        compiler_params=pltpu.CompilerParams(needs_layout_passes=False, use_tc_tiling_on_sc=False),
        scratch_types=[
            pltpu.VMEM_SHARED((NP, 16), f32),
            pltpu.VMEM_SHARED((NP, 16), f32),
            pltpu.VMEM((8, 128), i32),
            pltpu.VMEM((8, 128), i32),
            pltpu.VMEM((128, 16), f32),
        ],
    )
    def _kseg(y16_hbm, src2d_hbm, dst2d_hbm, z16_hbm, out_hbm,
              ytabS, accS, sbuf, dbuf, gbuf):
        cid = lax.axis_index("c")
        sid = lax.axis_index("s")
        wid = sid * 2 + cid
        rs = pl.multiple_of(sid * NPS, 128)
        pltpu.sync_copy(y16_hbm.at[pl.ds(rs, NPS)], ytabS.at[pl.ds(rs, NPS)])
        pltpu.sync_copy(z16_hbm.at[pl.ds(rs, NPS)], accS.at[pl.ds(rs, NPS)])
        plsc.subcore_barrier()
        for ch in range(NCH):
            r0 = pl.multiple_of(wid * (EPT // 128) + ch * (CH // 128), 8)
            pltpu.sync_copy(src2d_hbm.at[pl.ds(r0, 8)], sbuf)
            pltpu.sync_copy(dst2d_hbm.at[pl.ds(r0, 8)], dbuf)
            for r in range(8):
                pltpu.sync_copy(ytabS.at[sbuf.at[r]], gbuf)
                pltpu.sync_copy(gbuf, accS.at[dbuf.at[r]], add=True)
        plsc.subcore_barrier()
        pltpu.sync_copy(accS.at[pl.ds(rs, NPS)], out_hbm.at[cid, pl.ds(rs, NPS)])

    return _kseg


def _kseg_sum(y16, src2d, dst2d, z16):
    return _build_kseg()(y16, src2d, dst2d, z16)


@functools.lru_cache(maxsize=None)
def _build_k7():
    @functools.partial(
        pl.kernel,
        out_type=[
            jax.ShapeDtypeStruct((2, NP, HID), f32),
            jax.ShapeDtypeStruct((2, NP, 16), f32),
        ],
        mesh=_mesh(),
        compiler_params=pltpu.CompilerParams(needs_layout_passes=False, use_tc_tiling_on_sc=False),
        scratch_types=[
            pltpu.VMEM_SHARED((NP, HID), f32),
            pltpu.VMEM_SHARED((NP, 16), f32),
            pltpu.VMEM((NP * 4,), f32),
            pltpu.VMEM((8, 128), i32),
            pltpu.VMEM((8, 128), i32),
            pltpu.VMEM((CH,), f32),
            pltpu.VMEM((128, 16), f32),
            pltpu.VMEM((128, HID), f32),
            pltpu.VMEM((128, HID), f32),
            pltpu.VMEM((16,), f32),
            pltpu.SemaphoreType.DMA,
            pltpu.SemaphoreType.DMA,
            pltpu.SemaphoreType.DMA,
            pltpu.SemaphoreType.DMA,
        ],
    )
    def _k7(src2d_hbm, dst2d_hbm, ptab_hbm, xt_hbm, m16_hbm, z64_hbm, z16_hbm,
            nump_hbm, denp_hbm,
            numS, denS, ptab, sbuf, dbuf, exbuf, dpay, growA, growB, mv,
            gsemA, gsemB, ssemA, ssemB):
        cid = lax.axis_index("c")
        sid = lax.axis_index("s")
        wid = sid * 2 + cid
        rs = pl.multiple_of(sid * NPS, 128)
        pltpu.sync_copy(ptab_hbm, ptab)
        pltpu.sync_copy(m16_hbm, mv)
        pltpu.sync_copy(z64_hbm.at[pl.ds(rs, NPS)], numS.at[pl.ds(rs, NPS)])
        pltpu.sync_copy(z16_hbm.at[pl.ds(rs, NPS)], denS.at[pl.ds(rs, NPS)])
        pltpu.sync_copy(z16_hbm.at[pl.ds(0, 128)], dpay)
        plsc.subcore_barrier()
        M = mv[...]
        zl = jnp.zeros((16,), i32)
        il = lax.iota(i32, 16)

        def chbody(ch, carry):
            r0 = pl.multiple_of(wid * (EPT // 128) + ch * (CH // 128), 8)
            pltpu.sync_copy(src2d_hbm.at[pl.ds(r0, 8)], sbuf)
            pltpu.sync_copy(dst2d_hbm.at[pl.ds(r0, 8)], dbuf)
            for r in range(8):

                def exbody(j2, carry2, r=r):
                    sv = sbuf[r, pl.ds(j2 * 16, 16)]
                    dv = dbuf[r, pl.ds(j2 * 16, 16)]
                    sv4 = sv * 4
                    dv4 = dv * 4
                    a_s = plsc.load_gather(ptab, [sv4])
                    a_d = plsc.load_gather(ptab, [dv4 + 1])
                    m_s = plsc.load_gather(ptab, [sv4 + 3])
                    m_d = plsc.load_gather(ptab, [dv4 + 3])
                    t = a_s + a_d
                    lg = jnp.where(t >= 0, t, 0.2 * t)
                    ex = jnp.exp(lg - M)
                    exm = jnp.where(m_s * m_d > 0.5, ex, jnp.zeros((16,), f32))
                    exbuf[pl.ds(r * 128 + j2 * 16, 16)] = exm
                    return carry2

                lax.fori_loop(0, 8, exbody, 0)
            bufs = (growA, growB)
            gsems = (gsemA, gsemB)
            ssems = (ssemA, ssemB)
            gh = [None, None]
            sh = [None, None]
            gh[0] = pltpu.async_copy(xt_hbm.at[sbuf.at[0]], bufs[0], gsems[0])
            for r in range(8):
                b = r % 2
                if r + 1 < 8:
                    nb = (r + 1) % 2
                    if r >= 1:
                        sh[nb].wait()
                    gh[nb] = pltpu.async_copy(xt_hbm.at[sbuf.at[r + 1]],
                                              bufs[nb], gsems[nb])
                gh[b].wait()
                for jj in range(8):
                    v = exbuf[pl.ds(r * 128 + jj * 16, 16)]
                    plsc.store_scatter(dpay, [jj * 16 + il, zl], v)
                pltpu.sync_copy(dpay, denS.at[dbuf.at[r]], add=True)
                grow = bufs[b]

                def scbody(g, carry2, r=r, grow=grow):
                    ev = exbuf[pl.ds(r * 128 + g * 16, 16)]
                    for lane in range(16):
                        i = g * 16 + lane
                        e = ev[lane]
                        for q in range(4):
                            grow[i, pl.ds(q * 16, 16)] = e * grow[i, pl.ds(q * 16, 16)]
                    return carry2

                lax.fori_loop(0, 8, scbody, 0)
                sh[b] = pltpu.async_copy(grow, numS.at[dbuf.at[r]], ssems[b], add=True)
            sh[0].wait()
            sh[1].wait()
            return carry

        lax.fori_loop(0, NCH, chbody, 0)
        plsc.subcore_barrier()
        pltpu.sync_copy(numS.at[pl.ds(rs, NPS)], nump_hbm.at[cid, pl.ds(rs, NPS)])
        pltpu.sync_copy(denS.at[pl.ds(rs, NPS)], denp_hbm.at[cid, pl.ds(rs, NPS)])

    return _k7


def _k7_gat(src2d, dst2d, ptab, xt, m16, z64, z16):
    return _build_k7()(src2d, dst2d, ptab.reshape(NP * 4), xt, m16, z64, z16)


def _topk_thresholds(skey2d, k):
    t0 = jnp.where(jnp.sum((skey2d >= 0).astype(i32)) >= k, i32(0), i32(-2**31))

    def bit_body(bi, t):
        b = 30 - bi
        cand = t + (i32(1) << b)
        cnt = jnp.sum((skey2d >= cand).astype(i32))
        return jnp.where(cnt >= k, cand, t)

    t = lax.fori_loop(0, 31, bit_body, t0)
    gt = skey2d > t
    eq = skey2d == t
    need = k - jnp.sum(gt.astype(i32))
    pos = (lax.broadcasted_iota(i32, (80, 128), 0) * 128
           + lax.broadcasted_iota(i32, (80, 128), 1))

    def idx_body(bi, t2):
        b = 13 - bi
        cand = t2 | (i32(1) << b)
        cl = jnp.sum((eq & (pos < cand)).astype(i32))
        return jnp.where(cl < need, cand, t2)

    t2 = lax.fori_loop(0, 14, idx_body, i32(0))
    return t, t2


def _skey(score2d):
    bits = lax.bitcast_convert_type(score2d, i32)
    return jnp.where(bits >= 0, bits, bits ^ i32(0x7FFFFFFF))


def _k4_body(aggp_ref, cntp_ref, x_ref, pk_ref, x1_ref, y16_ref):
    cnt = cntp_ref[0, :, 0:1] + cntp_ref[1, :, 0:1]
    agg = (aggp_ref[0] + aggp_ref[1]) / jnp.maximum(cnt, 1.0)
    root = pk_ref[1:5]
    x1 = agg + jnp.dot(x_ref[...], root, preferred_element_type=f32) + pk_ref[0][None, :]
    x1_ref[...] = x1
    y1 = jnp.dot(x1, pk_ref[5][:, None], preferred_element_type=f32)
    y16_ref[...] = jnp.broadcast_to(y1, (NP, 16))


def _k4(aggp, cntp, xp4, pk):
    return pl.pallas_call(
        _k4_body,
        out_shape=[
            jax.ShapeDtypeStruct((NP, HID), f32),
            jax.ShapeDtypeStruct((NP, 16), f32),
        ],
    )(aggp, cntp, xp4, pk)


def _k6_body(x1_ref, nbsp_ref, pk_ref, gw_ref, xt_ref, ptab_ref, m_ref):
    x1 = x1_ref[...]
    nbs = nbsp_ref[0, :, 0:1] + nbsp_ref[1, :, 0:1]
    score = nbs + pk_ref[5, 0] + jnp.dot(x1, pk_ref[4][:, None], preferred_element_type=f32)
    score2d = score.reshape(80, 128)
    pos = (lax.broadcasted_iota(i32, (80, 128), 0) * 128
           + lax.broadcasted_iota(i32, (80, 128), 1))
    sk = jnp.where(pos < N, _skey(score2d), i32(-2**31))
    t, t2 = _topk_thresholds(sk, K1N)
    posc = lax.broadcasted_iota(i32, (NP, 1), 0)
    skc = jnp.where(posc < N, _skey(score), i32(-2**31))
    m1c = (skc > t) | ((skc == t) & (posc <= t2))
    xp = jnp.maximum(x1 * jnp.tanh(score), 0.0)
    xp = xp * (_INV * pk_ref[2][None, :]) + pk_ref[3][None, :]
    xt = jnp.dot(xp, gw_ref[...], preferred_element_type=f32)
    xt_ref[...] = xt
    asrc = jnp.dot(xt, pk_ref[0][:, None], preferred_element_type=f32)
    adst = jnp.dot(xt, pk_ref[1][:, None], preferred_element_type=f32)
    rowm = posc < N
    Ma = jnp.max(jnp.where(rowm, asrc, -3e38))
    Mb = jnp.max(jnp.where(rowm, adst, -3e38))
    M = _lrelu(Ma + Mb, 0.2)
    ssum = asrc + adst
    lg_s = jnp.where(ssum >= 0, ssum, 0.2 * ssum)
    exs = jnp.exp(lg_s - M)
    ptab_ref[...] = jnp.concatenate([asrc, adst, exs, m1c.astype(f32)], axis=1)
    m_ref[...] = jnp.full((8, 16), M, f32)


def _k6(x1, nbsp, pk, gw):
    return pl.pallas_call(
        _k6_body,
        out_shape=[
            jax.ShapeDtypeStruct((NP, HID), f32),
            jax.ShapeDtypeStruct((NP, 4), f32),
            jax.ShapeDtypeStruct((8, 16), f32),
        ],
    )(x1, nbsp, pk, gw)


def _k8_body(nump_ref, denp_ref, xt_ref, ptab_ref, pk_ref, x2_ref, y16_ref):
    xt = xt_ref[...]
    exs = ptab_ref[:, 2:3]
    m1 = ptab_ref[:, 3:4] > 0.5
    den = denp_ref[0, :, 0:1] + denp_ref[1, :, 0:1] + exs
    num = nump_ref[0] + nump_ref[1] + exs * xt
    x2 = num / den + pk_ref[0][None, :]
    x2 = jnp.where(m1, x2, 0.0)
    x2_ref[...] = x2
    y2 = jnp.dot(x2, pk_ref[1][:, None], preferred_element_type=f32)
    y2m = jnp.where(m1, y2, 0.0)
    y16_ref[...] = jnp.broadcast_to(y2m, (NP, 16))


def _k8(nump, denp, xt, ptab, pk):
    return pl.pallas_call(
        _k8_body,
        out_shape=[
            jax.ShapeDtypeStruct((NP, HID), f32),
            jax.ShapeDtypeStruct((NP, 16), f32),
        ],
    )(nump, denp, xt, ptab, pk)


def _k10_body(x2_ref, nb2p_ref, ptab_ref, pk_ref, wih_ref, whh_ref, sb_ref,
              fcw_ref, fcb_ref, out_ref):
    x2 = x2_ref[...]
    m1 = ptab_ref[:, 3:4] > 0.5
    nb2 = nb2p_ref[0, :, 0:1] + nb2p_ref[1, :, 0:1]
    score = nb2 + pk_ref[3, 0] + jnp.dot(x2, pk_ref[2][:, None], preferred_element_type=f32)
    score2d = score.reshape(80, 128)
    posc = lax.broadcasted_iota(i32, (NP, 1), 0)
    skc = jnp.where((posc < N) & m1, _skey(score), i32(-2**31))
    sk = skc.reshape(80, 128)
    t, t2 = _topk_thresholds(sk, K2N)
    m2 = (skc > t) | ((skc == t) & (posc <= t2))
    x3 = jnp.maximum(x2 * jnp.tanh(score), 0.0)
    x3 = x3 * (_INV * pk_ref[0][None, :]) + pk_ref[1][None, :]

    q_star = jnp.zeros((1, 2 * HID), f32)
    hC = jnp.zeros((1, HID), f32)
    cC = jnp.zeros((1, HID), f32)
    for _ in range(5):
        gates = (jnp.dot(q_star, wih_ref[...], preferred_element_type=f32)
                 + sb_ref[0][None, :]
                 + jnp.dot(hC, whh_ref[...], preferred_element_type=f32)
                 + sb_ref[1][None, :])
        ig = jax.nn.sigmoid(gates[:, 0:HID])
        fg = jax.nn.sigmoid(gates[:, HID:2 * HID])
        gg = jnp.tanh(gates[:, 2 * HID:3 * HID])
        og = jax.nn.sigmoid(gates[:, 3 * HID:4 * HID])
        cC = fg * cC + ig * gg
        hC = og * jnp.tanh(cC)
        eatt = jnp.sum(x3 * hC, axis=1, keepdims=True)
        eatt = jnp.where(m2, eatt, -3e38)
        mx = jnp.max(eatt)
        ex = jnp.where(m2, jnp.exp(eatt - mx), 0.0)
        aw = ex / jnp.sum(ex)
        r = jnp.sum(aw * x3, axis=0, keepdims=True)
        q_star = jnp.concatenate([hC, r], axis=1)

    res = jnp.dot(q_star, fcw_ref[...], preferred_element_type=f32) + fcb_ref[0][None, :]
    out_ref[...] = jnp.zeros((8, 128), f32)
    out_ref[0:1, 0:32] = res


def _k10(x2, nb2p, ptab, pk, wih, whh, sb, fcw, fcb):
    return pl.pallas_call(
        _k10_body,
        out_shape=jax.ShapeDtypeStruct((8, 128), f32),
    )(x2, nb2p, ptab, pk, wih, whh, sb, fcw, fcb)


def kernel(x, edge_attr, params, edge_index, batch):
    p = params
    src = edge_index[0]
    dst = edge_index[1]
    srcp = jnp.concatenate([src, jnp.zeros((EP - E,), i32)])
    dstp = jnp.concatenate([dst, jnp.full((EP - E,), NP - 1, i32)])
    src2d = srcp.reshape(EP // 128, 128)
    dst2d = dstp.reshape(EP // 128, 128)
    apad = jnp.pad(edge_attr[:, 0], (0, EP - E))
    a3 = apad.reshape(NBE2, 32, 128)
    a2d = apad.reshape(EP // 128, 128)
    xp4 = jnp.pad(x, ((0, NP - N), (0, 0)))
    z64 = jnp.zeros((NP, HID), f32)
    z16 = jnp.zeros((NP, 16), f32)
    ones16 = jnp.ones((128, 16), f32)

    alpha = p['ea_et_W'][0] @ p['ea_st_W'][:, 0]
    beta = p['ea_et_b'] @ p['ea_st_W'][:, 0] + p['ea_st_b'][0]
    A = p['em_W1'][0] * _INV * p['em_bn_g']
    C = (p['em_b1'] * _INV) * p['em_bn_g'] + p['em_bn_b']
    B2 = p['em_b2'].reshape(NF, HID)
    pk2 = jnp.zeros((8, HID), f32)
    pk2 = pk2.at[0].set(A).at[1].set(C).at[2:6].set(B2)
    pk2 = pk2.at[6, 0].set(alpha).at[6, 1].set(beta)

    pk4 = jnp.zeros((8, HID), f32)
    pk4 = pk4.at[0].set(p['c1_b']).at[1:5].set(p['c1_root']).at[5].set(p['p1_rel_W'][:, 0])

    pk6 = jnp.zeros((8, HID), f32)
    pk6 = (pk6.at[0].set(p['g_asrc']).at[1].set(p['g_adst'])
              .at[2].set(p['bn1_g']).at[3].set(p['bn1_b'])
              .at[4].set(p['p1_root_W'][:, 0]).at[5, 0].set(p['p1_rel_b'][0]))

    pk8 = jnp.zeros((8, HID), f32)
    pk8 = pk8.at[0].set(p['g_b']).at[1].set(p['p2_rel_W'][:, 0])

    pk10 = jnp.zeros((8, HID), f32)
    pk10 = (pk10.at[0].set(p['bn2_g']).at[1].set(p['bn2_b'])
                .at[2].set(p['p2_root_W'][:, 0]).at[3, 0].set(p['p2_rel_b'][0]))
    wih = p['s2s_Wih'].T
    whh = p['s2s_Whh'].T
    sb = jnp.zeros((8, 4 * HID), f32)
    sb = sb.at[0].set(p['s2s_bih']).at[1].set(p['s2s_bhh'])
    fcw = p['fc_W']
    fcb = jnp.zeros((8, 32), f32).at[0].set(p['fc_b'])

    xs = _k1_gather(xp4.reshape(NP * 4), srcp)
    stats = _k2a_stats(a2d, pk2)
    msg = _k2_msg(a3, xs, pk2, p['em_W2'], stats)
    aggp, cntp = _k3_scatter(msg, dst2d, z64, z16, ones16)
    x1, y16 = _k4(aggp, cntp, xp4, pk4)
    nbsp = _kseg_sum(y16, src2d, dst2d, z16)
    xt, ptab, m8 = _k6(x1, nbsp, pk6, p['g_W'])
    m16 = m8.reshape(128)[0:16]
    nump, denp = _k7_gat(src2d, dst2d, ptab, xt, m16, z64, z16)
    x2, y216 = _k8(nump, denp, xt, ptab, pk8)
    nb2p = _kseg_sum(y216, src2d, dst2d, z16)
    out = _k10(x2, nb2p, ptab, pk10, wih, whh, sb, fcw, fcb)
    return out[0:1, 0:32]


# speedup vs baseline: 11.6432x; 1.0113x over previous
import functools
import math

import jax
import jax.numpy as jnp
from jax import lax
from jax.experimental import pallas as pl
from jax.experimental.pallas import tpu as pltpu
from jax.experimental.pallas import tpu_sc as plsc

N = 10000
NP = 10240
E = 160000
EP = 163840
HID = 64
NF = 4
K1N = (N + 1) // 2
K2N = (K1N + 1) // 2
NW = 32
EPT = EP // NW
CH = 1024
NCH = EPT // CH
BE = 2048
NBE = EP // BE
NPS = NP // 16
_INV = 1.0 / math.sqrt(1.0 + 1e-5)

f32 = jnp.float32
i32 = jnp.int32


def _mesh():
    return plsc.VectorSubcoreMesh(core_axis_name="c", subcore_axis_name="s")


def _lrelu(x, s=0.01):
    return jnp.where(x >= 0, x, s * x)


@functools.lru_cache(maxsize=None)
def _build_k1():
    @functools.partial(
        pl.kernel,
        out_type=jax.ShapeDtypeStruct((8, EP), f32),
        mesh=_mesh(),
        compiler_params=pltpu.CompilerParams(needs_layout_passes=False, use_tc_tiling_on_sc=False),
        scratch_types=[
            pltpu.VMEM((NP * 4,), f32),
            pltpu.VMEM((EPT,), i32),
            pltpu.VMEM((4, EPT), f32),
        ],
    )
    def _k1(x_hbm, src_hbm, xs_hbm, xtab, sidx, xsb):
        cid = lax.axis_index("c")
        sid = lax.axis_index("s")
        wid = sid * 2 + cid
        base = pl.multiple_of(wid * EPT, 128)
        pltpu.sync_copy(x_hbm, xtab)
        pltpu.sync_copy(src_hbm.at[pl.ds(base, EPT)], sidx)

        def body(j, carry):
            iv = sidx[pl.ds(j * 16, 16)]
            iv4 = iv * 4
            for f in range(4):
                g = plsc.load_gather(xtab, [iv4 + f])
                xsb[f, pl.ds(j * 16, 16)] = g
            return carry

        lax.fori_loop(0, EPT // 16, body, 0)
        for f in range(4):
            pltpu.sync_copy(xsb.at[f], xs_hbm.at[f, pl.ds(base, EPT)])

    return _k1


def _k1_gather(xp4, srcp):
    return _build_k1()(xp4, srcp)


def _k2a_body(a_ref, pk_ref, stat_ref):
    a = a_ref[...]
    alpha = pk_ref[6, 0]
    beta = pk_ref[6, 1]
    s_l = _lrelu(alpha * a + beta)
    pos = (lax.broadcasted_iota(i32, (EP // 128, 128), 0) * 128
           + lax.broadcasted_iota(i32, (EP // 128, 128), 1))
    valid = pos < E
    mb = jnp.max(jnp.where(valid, s_l, -3e38))
    sb = jnp.sum(jnp.where(valid, jnp.exp(s_l - mb), 0.0))
    rr = lax.broadcasted_iota(i32, (8, 128), 0)
    cc = lax.broadcasted_iota(i32, (8, 128), 1)
    stat_ref[...] = (jnp.where((rr == 0) & (cc == 0), mb, 0.0)
                     + jnp.where((rr == 0) & (cc == 1), sb, 0.0))


def _k2a_stats(a2d, pk):
    return pl.pallas_call(
        _k2a_body,
        out_shape=jax.ShapeDtypeStruct((8, 128), f32),
    )(a2d, pk)


BE2 = 4096
NBE2 = EP // BE2


def _k2_body(a_ref, xs_ref, pk_ref, w2_ref, stat_ref, msg_ref):
    i = pl.program_id(0)
    del i
    ab = a_ref[0].reshape(BE2)
    alpha = pk_ref[6, 0]
    beta = pk_ref[6, 1]
    s_l = _lrelu(alpha * ab + beta)
    m0 = stat_ref[0, 0]
    s0 = stat_ref[0, 1]
    sm = jnp.exp(s_l - m0) / s0
    ea = _lrelu(ab * sm)
    A = pk_ref[0]
    C = pk_ref[1]
    h = _lrelu(ea[:, None] * A[None, :] + C[None, :])
    acc = jnp.zeros((BE2, HID), f32)
    for fdim in range(4):
        wf = w2_ref[:, fdim * HID:(fdim + 1) * HID]
        hf = jnp.dot(h, wf, preferred_element_type=f32) + pk_ref[2 + fdim][None, :]
        acc = acc + xs_ref[fdim][:, None] * hf
    msg_ref[...] = acc


def _k2_msg(a3, xs, pk, w2, stats):
    return pl.pallas_call(
        _k2_body,
        grid=(NBE2,),
        in_specs=[
            pl.BlockSpec((1, 32, 128), lambda i: (i, 0, 0)),
            pl.BlockSpec((8, BE2), lambda i: (0, i)),
            pl.BlockSpec((8, HID), lambda i: (0, 0)),
            pl.BlockSpec((HID, NF * HID), lambda i: (0, 0)),
            pl.BlockSpec((8, 128), lambda i: (0, 0)),
        ],
        out_specs=pl.BlockSpec((BE2, HID), lambda i: (i, 0)),
        out_shape=jax.ShapeDtypeStruct((EP, HID), f32),
    )(a3, xs, pk, w2, stats)


CH3 = 512
NCH3 = EPT // CH3


@functools.lru_cache(maxsize=None)
def _build_k3():
    @functools.partial(
        pl.kernel,
        out_type=[
            jax.ShapeDtypeStruct((2, NP, HID), f32),
            jax.ShapeDtypeStruct((2, NP, 16), f32),
        ],
        mesh=_mesh(),
        compiler_params=pltpu.CompilerParams(needs_layout_passes=False, use_tc_tiling_on_sc=False),
        scratch_types=[
            pltpu.VMEM_SHARED((NP, HID), f32),
            pltpu.VMEM_SHARED((NP, 16), f32),
            pltpu.VMEM((4, 128), i32),
            pltpu.VMEM((4, 128), i32),
            pltpu.VMEM((CH3, HID), f32),
            pltpu.VMEM((CH3, HID), f32),
            pltpu.VMEM((128, 16), f32),
            pltpu.SemaphoreType.DMA,
            pltpu.SemaphoreType.DMA,
        ],
    )
    def _k3(msg_hbm, dst2d_hbm, z64_hbm, z16_hbm, ones_hbm,
            aggp_hbm, cntp_hbm, aggS, cntS, dbufA, dbufB, mbufA, mbufB, onesv,
            semA, semB):
        cid = lax.axis_index("c")
        sid = lax.axis_index("s")
        wid = sid * 2 + cid
        rs = pl.multiple_of(sid * NPS, 128)
        pltpu.sync_copy(z64_hbm.at[pl.ds(rs, NPS)], aggS.at[pl.ds(rs, NPS)])
        pltpu.sync_copy(z16_hbm.at[pl.ds(rs, NPS)], cntS.at[pl.ds(rs, NPS)])
        pltpu.sync_copy(ones_hbm, onesv)
        plsc.subcore_barrier()
        dbufs = (dbufA, dbufB)
        mbufs = (mbufA, mbufB)
        sems = (semA, semB)
        hand = [None, None]

        def _issue(ch, b):
            e0 = pl.multiple_of(wid * EPT + ch * CH3, 128)
            r0 = pl.multiple_of(wid * (EPT // 128) + ch * (CH3 // 128), 4)
            h1 = pltpu.async_copy(dst2d_hbm.at[pl.ds(r0, 4)], dbufs[b], sems[b])
            h2 = pltpu.async_copy(msg_hbm.at[pl.ds(e0, CH3)], mbufs[b], sems[b])
            return (h1, h2)

        hand[0] = _issue(0, 0)
        for ch in range(NCH3):
            b = ch % 2
            hand[b][0].wait()
            hand[b][1].wait()
            if ch + 1 < NCH3:
                hand[1 - b] = _issue(ch + 1, 1 - b)
            for r in range(4):
                pltpu.sync_copy(mbufs[b].at[pl.ds(r * 128, 128)],
                                aggS.at[dbufs[b].at[r]], add=True)
                pltpu.sync_copy(onesv, cntS.at[dbufs[b].at[r]], add=True)
        plsc.subcore_barrier()
        pltpu.sync_copy(aggS.at[pl.ds(rs, NPS)], aggp_hbm.at[cid, pl.ds(rs, NPS)])
        pltpu.sync_copy(cntS.at[pl.ds(rs, NPS)], cntp_hbm.at[cid, pl.ds(rs, NPS)])

    return _k3


def _k3_scatter(msg, dst2d, z64, z16, ones16):
    return _build_k3()(msg, dst2d, z64, z16, ones16)


@functools.lru_cache(maxsize=None)
def _build_kseg():
    @functools.partial(
        pl.kernel,
        out_type=jax.ShapeDtypeStruct((2, NP, 16), f32),
        mesh=_mesh(),
        compiler_params=pltpu.CompilerParams(needs_layout_passes=False, use_tc_tiling_on_sc=False),
        scratch_types=[
            pltpu.VMEM_SHARED((NP, 16), f32),
            pltpu.VMEM_SHARED((NP, 16), f32),
            pltpu.VMEM((8, 128), i32),
            pltpu.VMEM((8, 128), i32),
            pltpu.VMEM((128, 16), f32),
        ],
    )
    def _kseg(y16_hbm, src2d_hbm, dst2d_hbm, z16_hbm, out_hbm,
              ytabS, accS, sbuf, dbuf, gbuf):
        cid = lax.axis_index("c")
        sid = lax.axis_index("s")
        wid = sid * 2 + cid
        rs = pl.multiple_of(sid * NPS, 128)
        pltpu.sync_copy(y16_hbm.at[pl.ds(rs, NPS)], ytabS.at[pl.ds(rs, NPS)])
        pltpu.sync_copy(z16_hbm.at[pl.ds(rs, NPS)], accS.at[pl.ds(rs, NPS)])
        plsc.subcore_barrier()
        for ch in range(NCH):
            r0 = pl.multiple_of(wid * (EPT // 128) + ch * (CH // 128), 8)
            pltpu.sync_copy(src2d_hbm.at[pl.ds(r0, 8)], sbuf)
            pltpu.sync_copy(dst2d_hbm.at[pl.ds(r0, 8)], dbuf)
            for r in range(8):
                pltpu.sync_copy(ytabS.at[sbuf.at[r]], gbuf)
                pltpu.sync_copy(gbuf, accS.at[dbuf.at[r]], add=True)
        plsc.subcore_barrier()
        pltpu.sync_copy(accS.at[pl.ds(rs, NPS)], out_hbm.at[cid, pl.ds(rs, NPS)])

    return _kseg


def _kseg_sum(y16, src2d, dst2d, z16):
    return _build_kseg()(y16, src2d, dst2d, z16)


@functools.lru_cache(maxsize=None)
def _build_k7():
    @functools.partial(
        pl.kernel,
        out_type=[
            jax.ShapeDtypeStruct((2, NP, HID), f32),
            jax.ShapeDtypeStruct((2, NP, 16), f32),
        ],
        mesh=_mesh(),
        compiler_params=pltpu.CompilerParams(needs_layout_passes=False, use_tc_tiling_on_sc=False),
        scratch_types=[
            pltpu.VMEM_SHARED((NP, HID), f32),
            pltpu.VMEM_SHARED((NP, 16), f32),
            pltpu.VMEM((NP * 4,), f32),
            pltpu.VMEM((8, 128), i32),
            pltpu.VMEM((8, 128), i32),
            pltpu.VMEM((CH,), f32),
            pltpu.VMEM((128, 16), f32),
            pltpu.VMEM((128, HID), f32),
            pltpu.VMEM((128, HID), f32),
            pltpu.VMEM((16,), f32),
            pltpu.SemaphoreType.DMA,
            pltpu.SemaphoreType.DMA,
            pltpu.SemaphoreType.DMA,
            pltpu.SemaphoreType.DMA,
        ],
    )
    def _k7(src2d_hbm, dst2d_hbm, ptab_hbm, xt_hbm, m16_hbm, z64_hbm, z16_hbm,
            nump_hbm, denp_hbm,
            numS, denS, ptab, sbuf, dbuf, exbuf, dpay, growA, growB, mv,
            gsemA, gsemB, ssemA, ssemB):
        cid = lax.axis_index("c")
        sid = lax.axis_index("s")
        wid = sid * 2 + cid
        rs = pl.multiple_of(sid * NPS, 128)
        pltpu.sync_copy(ptab_hbm, ptab)
        pltpu.sync_copy(m16_hbm, mv)
        pltpu.sync_copy(z64_hbm.at[pl.ds(rs, NPS)], numS.at[pl.ds(rs, NPS)])
        pltpu.sync_copy(z16_hbm.at[pl.ds(rs, NPS)], denS.at[pl.ds(rs, NPS)])
        pltpu.sync_copy(z16_hbm.at[pl.ds(0, 128)], dpay)
        plsc.subcore_barrier()
        M = mv[...]
        zl = jnp.zeros((16,), i32)
        il = lax.iota(i32, 16)

        def chbody(ch, carry):
            r0 = pl.multiple_of(wid * (EPT // 128) + ch * (CH // 128), 8)
            pltpu.sync_copy(src2d_hbm.at[pl.ds(r0, 8)], sbuf)
            pltpu.sync_copy(dst2d_hbm.at[pl.ds(r0, 8)], dbuf)
            for r in range(8):

                def exbody(j2, carry2, r=r):
                    sv = sbuf[r, pl.ds(j2 * 16, 16)]
                    dv = dbuf[r, pl.ds(j2 * 16, 16)]
                    sv4 = sv * 4
                    dv4 = dv * 4
                    a_s = plsc.load_gather(ptab, [sv4])
                    a_d = plsc.load_gather(ptab, [dv4 + 1])
                    m_s = plsc.load_gather(ptab, [sv4 + 3])
                    m_d = plsc.load_gather(ptab, [dv4 + 3])
                    t = a_s + a_d
                    lg = jnp.where(t >= 0, t, 0.2 * t)
                    ex = jnp.exp(lg - M)
                    exm = jnp.where(m_s * m_d > 0.5, ex, jnp.zeros((16,), f32))
                    exbuf[pl.ds(r * 128 + j2 * 16, 16)] = exm
                    return carry2

                lax.fori_loop(0, 8, exbody, 0)
            bufs = (growA, growB)
            gsems = (gsemA, gsemB)
            ssems = (ssemA, ssemB)
            gh = [None, None]
            sh = [None, None]
            gh[0] = pltpu.async_copy(xt_hbm.at[sbuf.at[0]], bufs[0], gsems[0])
            for r in range(8):
                b = r % 2
                if r + 1 < 8:
                    nb = (r + 1) % 2
                    if r >= 1:
                        sh[nb].wait()
                    gh[nb] = pltpu.async_copy(xt_hbm.at[sbuf.at[r + 1]],
                                              bufs[nb], gsems[nb])
                gh[b].wait()
                for jj in range(8):
                    v = exbuf[pl.ds(r * 128 + jj * 16, 16)]
                    plsc.store_scatter(dpay, [jj * 16 + il, zl], v)
                pltpu.sync_copy(dpay, denS.at[dbuf.at[r]], add=True)
                grow = bufs[b]

                def scbody(g, carry2, r=r, grow=grow):
                    ev = exbuf[pl.ds(r * 128 + g * 16, 16)]
                    for lane in range(16):
                        i = g * 16 + lane
                        e = ev[lane]
                        for q in range(4):
                            grow[i, pl.ds(q * 16, 16)] = e * grow[i, pl.ds(q * 16, 16)]
                    return carry2

                lax.fori_loop(0, 8, scbody, 0)
                sh[b] = pltpu.async_copy(grow, numS.at[dbuf.at[r]], ssems[b], add=True)
            sh[0].wait()
            sh[1].wait()
            return carry

        lax.fori_loop(0, NCH, chbody, 0)
        plsc.subcore_barrier()
        pltpu.sync_copy(numS.at[pl.ds(rs, NPS)], nump_hbm.at[cid, pl.ds(rs, NPS)])
        pltpu.sync_copy(denS.at[pl.ds(rs, NPS)], denp_hbm.at[cid, pl.ds(rs, NPS)])

    return _k7


def _k7_gat(src2d, dst2d, ptab, xt, m16, z64, z16):
    return _build_k7()(src2d, dst2d, ptab.reshape(NP * 4), xt, m16, z64, z16)


def _topk_thresholds(skey2d, k):
    t0 = jnp.where(jnp.sum((skey2d >= 0).astype(i32)) >= k, i32(0), i32(-2**31))

    def bit_body(bi, t):
        b = 30 - bi
        cand = t + (i32(1) << b)
        cnt = jnp.sum((skey2d >= cand).astype(i32))
        return jnp.where(cnt >= k, cand, t)

    t = lax.fori_loop(0, 31, bit_body, t0)
    gt = skey2d > t
    eq = skey2d == t
    need = k - jnp.sum(gt.astype(i32))
    pos = (lax.broadcasted_iota(i32, (80, 128), 0) * 128
           + lax.broadcasted_iota(i32, (80, 128), 1))

    def idx_body(bi, t2):
        b = 13 - bi
        cand = t2 | (i32(1) << b)
        cl = jnp.sum((eq & (pos < cand)).astype(i32))
        return jnp.where(cl < need, cand, t2)

    t2 = lax.fori_loop(0, 14, idx_body, i32(0))
    return t, t2


def _skey(score2d):
    bits = lax.bitcast_convert_type(score2d, i32)
    return jnp.where(bits >= 0, bits, bits ^ i32(0x7FFFFFFF))


def _k4_body(aggp_ref, cntp_ref, x_ref, pk_ref, x1_ref, y16_ref):
    cnt = cntp_ref[0, :, 0:1] + cntp_ref[1, :, 0:1]
    agg = (aggp_ref[0] + aggp_ref[1]) / jnp.maximum(cnt, 1.0)
    root = pk_ref[1:5]
    x1 = agg + jnp.dot(x_ref[...], root, preferred_element_type=f32) + pk_ref[0][None, :]
    x1_ref[...] = x1
    y1 = jnp.dot(x1, pk_ref[5][:, None], preferred_element_type=f32)
    y16_ref[...] = jnp.broadcast_to(y1, (NP, 16))


def _k4(aggp, cntp, xp4, pk):
    return pl.pallas_call(
        _k4_body,
        out_shape=[
            jax.ShapeDtypeStruct((NP, HID), f32),
            jax.ShapeDtypeStruct((NP, 16), f32),
        ],
    )(aggp, cntp, xp4, pk)


def _k6_body(x1_ref, nbsp_ref, pk_ref, gw_ref, xt_ref, ptab_ref, m_ref):
    x1 = x1_ref[...]
    nbs = nbsp_ref[0, :, 0:1] + nbsp_ref[1, :, 0:1]
    score = nbs + pk_ref[5, 0] + jnp.dot(x1, pk_ref[4][:, None], preferred_element_type=f32)
    score2d = score.reshape(80, 128)
    pos = (lax.broadcasted_iota(i32, (80, 128), 0) * 128
           + lax.broadcasted_iota(i32, (80, 128), 1))
    sk = jnp.where(pos < N, _skey(score2d), i32(-2**31))
    t, t2 = _topk_thresholds(sk, K1N)
    posc = lax.broadcasted_iota(i32, (NP, 1), 0)
    skc = jnp.where(posc < N, _skey(score), i32(-2**31))
    m1c = (skc > t) | ((skc == t) & (posc <= t2))
    xp = jnp.maximum(x1 * jnp.tanh(score), 0.0)
    xp = xp * (_INV * pk_ref[2][None, :]) + pk_ref[3][None, :]
    xt = jnp.dot(xp, gw_ref[...], preferred_element_type=f32)
    xt_ref[...] = xt
    asrc = jnp.dot(xt, pk_ref[0][:, None], preferred_element_type=f32)
    adst = jnp.dot(xt, pk_ref[1][:, None], preferred_element_type=f32)
    rowm = posc < N
    Ma = jnp.max(jnp.where(rowm, asrc, -3e38))
    Mb = jnp.max(jnp.where(rowm, adst, -3e38))
    M = _lrelu(Ma + Mb, 0.2)
    ssum = asrc + adst
    lg_s = jnp.where(ssum >= 0, ssum, 0.2 * ssum)
    exs = jnp.exp(lg_s - M)
    ptab_ref[...] = jnp.concatenate([asrc, adst, exs, m1c.astype(f32)], axis=1)
    m_ref[...] = jnp.full((8, 16), M, f32)


def _k6(x1, nbsp, pk, gw):
    return pl.pallas_call(
        _k6_body,
        out_shape=[
            jax.ShapeDtypeStruct((NP, HID), f32),
            jax.ShapeDtypeStruct((NP, 4), f32),
            jax.ShapeDtypeStruct((8, 16), f32),
        ],
    )(x1, nbsp, pk, gw)


def _k8_body(nump_ref, denp_ref, xt_ref, ptab_ref, pk_ref, x2_ref, y16_ref):
    xt = xt_ref[...]
    exs = ptab_ref[:, 2:3]
    m1 = ptab_ref[:, 3:4] > 0.5
    den = denp_ref[0, :, 0:1] + denp_ref[1, :, 0:1] + exs
    num = nump_ref[0] + nump_ref[1] + exs * xt
    x2 = num / den + pk_ref[0][None, :]
    x2 = jnp.where(m1, x2, 0.0)
    x2_ref[...] = x2
    y2 = jnp.dot(x2, pk_ref[1][:, None], preferred_element_type=f32)
    y2m = jnp.where(m1, y2, 0.0)
    y16_ref[...] = jnp.broadcast_to(y2m, (NP, 16))


def _k8(nump, denp, xt, ptab, pk):
    return pl.pallas_call(
        _k8_body,
        out_shape=[
            jax.ShapeDtypeStruct((NP, HID), f32),
            jax.ShapeDtypeStruct((NP, 16), f32),
        ],
    )(nump, denp, xt, ptab, pk)


def _k10_body(x2_ref, nb2p_ref, ptab_ref, pk_ref, wih_ref, whh_ref, sb_ref,
              fcw_ref, fcb_ref, out_ref):
    x2 = x2_ref[...]
    m1 = ptab_ref[:, 3:4] > 0.5
    nb2 = nb2p_ref[0, :, 0:1] + nb2p_ref[1, :, 0:1]
    score = nb2 + pk_ref[3, 0] + jnp.dot(x2, pk_ref[2][:, None], preferred_element_type=f32)
    score2d = score.reshape(80, 128)
    posc = lax.broadcasted_iota(i32, (NP, 1), 0)
    skc = jnp.where((posc < N) & m1, _skey(score), i32(-2**31))
    sk = skc.reshape(80, 128)
    t, t2 = _topk_thresholds(sk, K2N)
    m2 = (skc > t) | ((skc == t) & (posc <= t2))
    x3 = jnp.maximum(x2 * jnp.tanh(score), 0.0)
    x3 = x3 * (_INV * pk_ref[0][None, :]) + pk_ref[1][None, :]

    q_star = jnp.zeros((1, 2 * HID), f32)
    hC = jnp.zeros((1, HID), f32)
    cC = jnp.zeros((1, HID), f32)
    for _ in range(5):
        gates = (jnp.dot(q_star, wih_ref[...], preferred_element_type=f32)
                 + sb_ref[0][None, :]
                 + jnp.dot(hC, whh_ref[...], preferred_element_type=f32)
                 + sb_ref[1][None, :])
        ig = jax.nn.sigmoid(gates[:, 0:HID])
        fg = jax.nn.sigmoid(gates[:, HID:2 * HID])
        gg = jnp.tanh(gates[:, 2 * HID:3 * HID])
        og = jax.nn.sigmoid(gates[:, 3 * HID:4 * HID])
        cC = fg * cC + ig * gg
        hC = og * jnp.tanh(cC)
        eatt = jnp.sum(x3 * hC, axis=1, keepdims=True)
        eatt = jnp.where(m2, eatt, -3e38)
        mx = jnp.max(eatt)
        ex = jnp.where(m2, jnp.exp(eatt - mx), 0.0)
        aw = ex / jnp.sum(ex)
        r = jnp.sum(aw * x3, axis=0, keepdims=True)
        q_star = jnp.concatenate([hC, r], axis=1)

    res = jnp.dot(q_star, fcw_ref[...], preferred_element_type=f32) + fcb_ref[0][None, :]
    out_ref[...] = jnp.zeros((8, 128), f32)
    out_ref[0:1, 0:32] = res


def _k10(x2, nb2p, ptab, pk, wih, whh, sb, fcw, fcb):
    return pl.pallas_call(
        _k10_body,
        out_shape=jax.ShapeDtypeStruct((8, 128), f32),
    )(x2, nb2p, ptab, pk, wih, whh, sb, fcw, fcb)


def kernel(x, edge_attr, params, edge_index, batch):
    p = params
    src = edge_index[0]
    dst = edge_index[1]
    srcp = jnp.concatenate([src, jnp.zeros((EP - E,), i32)])
    dstp = jnp.concatenate([dst, jnp.full((EP - E,), NP - 1, i32)])
    src2d = srcp.reshape(EP // 128, 128)
    dst2d = dstp.reshape(EP // 128, 128)
    apad = jnp.pad(edge_attr[:, 0], (0, EP - E))
    a3 = apad.reshape(NBE2, 32, 128)
    a2d = apad.reshape(EP // 128, 128)
    xp4 = jnp.pad(x, ((0, NP - N), (0, 0)))
    z64 = jnp.zeros((NP, HID), f32)
    z16 = jnp.zeros((NP, 16), f32)
    ones16 = jnp.ones((128, 16), f32)

    alpha = p['ea_et_W'][0] @ p['ea_st_W'][:, 0]
    beta = p['ea_et_b'] @ p['ea_st_W'][:, 0] + p['ea_st_b'][0]
    A = p['em_W1'][0] * _INV * p['em_bn_g']
    C = (p['em_b1'] * _INV) * p['em_bn_g'] + p['em_bn_b']
    B2 = p['em_b2'].reshape(NF, HID)
    pk2 = jnp.zeros((8, HID), f32)
    pk2 = pk2.at[0].set(A).at[1].set(C).at[2:6].set(B2)
    pk2 = pk2.at[6, 0].set(alpha).at[6, 1].set(beta)

    pk4 = jnp.zeros((8, HID), f32)
    pk4 = pk4.at[0].set(p['c1_b']).at[1:5].set(p['c1_root']).at[5].set(p['p1_rel_W'][:, 0])

    pk6 = jnp.zeros((8, HID), f32)
    pk6 = (pk6.at[0].set(p['g_asrc']).at[1].set(p['g_adst'])
              .at[2].set(p['bn1_g']).at[3].set(p['bn1_b'])
              .at[4].set(p['p1_root_W'][:, 0]).at[5, 0].set(p['p1_rel_b'][0]))

    pk8 = jnp.zeros((8, HID), f32)
    pk8 = pk8.at[0].set(p['g_b']).at[1].set(p['p2_rel_W'][:, 0])

    pk10 = jnp.zeros((8, HID), f32)
    pk10 = (pk10.at[0].set(p['bn2_g']).at[1].set(p['bn2_b'])
                .at[2].set(p['p2_root_W'][:, 0]).at[3, 0].set(p['p2_rel_b'][0]))
    wih = p['s2s_Wih'].T
    whh = p['s2s_Whh'].T
    sb = jnp.zeros((8, 4 * HID), f32)
    sb = sb.at[0].set(p['s2s_bih']).at[1].set(p['s2s_bhh'])
    fcw = p['fc_W']
    fcb = jnp.zeros((8, 32), f32).at[0].set(p['fc_b'])

    xs = _k1_gather(xp4.reshape(NP * 4), srcp)
    stats = _k2a_stats(a2d, pk2)
    msg = _k2_msg(a3, xs, pk2, p['em_W2'], stats)
    aggp, cntp = _k3_scatter(msg, dst2d, z64, z16, ones16)
    x1, y16 = _k4(aggp, cntp, xp4, pk4)
    nbsp = _kseg_sum(y16, src2d, dst2d, z16)
    xt, ptab, m8 = _k6(x1, nbsp, pk6, p['g_W'])
    m16 = m8.reshape(128)[0:16]
    nump, denp = _k7_gat(src2d, dst2d, ptab, xt, m16, z64, z16)
    x2, y216 = _k8(nump, denp, xt, ptab, pk8)
    nb2p = _kseg_sum(y216, src2d, dst2d, z16)
    out = _k10(x2, nb2p, ptab, pk10, wih, whh, sb, fcw, fcb)
    return out[0:1, 0:32]


# speedup vs baseline: 11.7362x; 1.0080x over previous
import functools
import math

import jax
import jax.numpy as jnp
from jax import lax
from jax.experimental import pallas as pl
from jax.experimental.pallas import tpu as pltpu
from jax.experimental.pallas import tpu_sc as plsc

N = 10000
NP = 10240
E = 160000
EP = 163840
HID = 64
NF = 4
K1N = (N + 1) // 2
K2N = (K1N + 1) // 2
NW = 32
EPT = EP // NW
CH = 1024
NCH = EPT // CH
BE = 2048
NBE = EP // BE
NPS = NP // 16
_INV = 1.0 / math.sqrt(1.0 + 1e-5)

f32 = jnp.float32
i32 = jnp.int32


def _mesh():
    return plsc.VectorSubcoreMesh(core_axis_name="c", subcore_axis_name="s")


def _lrelu(x, s=0.01):
    return jnp.where(x >= 0, x, s * x)


@functools.lru_cache(maxsize=None)
def _build_k1():
    @functools.partial(
        pl.kernel,
        out_type=jax.ShapeDtypeStruct((8, EP), f32),
        mesh=_mesh(),
        compiler_params=pltpu.CompilerParams(needs_layout_passes=False, use_tc_tiling_on_sc=False),
        scratch_types=[
            pltpu.VMEM((NP * 4,), f32),
            pltpu.VMEM((EPT,), i32),
            pltpu.VMEM((4, EPT), f32),
        ],
    )
    def _k1(x_hbm, src_hbm, xs_hbm, xtab, sidx, xsb):
        cid = lax.axis_index("c")
        sid = lax.axis_index("s")
        wid = sid * 2 + cid
        base = pl.multiple_of(wid * EPT, 128)
        pltpu.sync_copy(x_hbm, xtab)
        pltpu.sync_copy(src_hbm.at[pl.ds(base, EPT)], sidx)

        def body(j, carry):
            iv = sidx[pl.ds(j * 16, 16)]
            iv4 = iv * 4
            for f in range(4):
                g = plsc.load_gather(xtab, [iv4 + f])
                xsb[f, pl.ds(j * 16, 16)] = g
            return carry

        lax.fori_loop(0, EPT // 16, body, 0)
        for f in range(4):
            pltpu.sync_copy(xsb.at[f], xs_hbm.at[f, pl.ds(base, EPT)])

    return _k1


def _k1_gather(xp4, srcp):
    return _build_k1()(xp4, srcp)


def _k2a_body(a_ref, pk_ref, stat_ref):
    a = a_ref[...]
    alpha = pk_ref[6, 0]
    beta = pk_ref[6, 1]
    s_l = _lrelu(alpha * a + beta)
    pos = (lax.broadcasted_iota(i32, (EP // 128, 128), 0) * 128
           + lax.broadcasted_iota(i32, (EP // 128, 128), 1))
    valid = pos < E
    mb = jnp.max(jnp.where(valid, s_l, -3e38))
    sb = jnp.sum(jnp.where(valid, jnp.exp(s_l - mb), 0.0))
    rr = lax.broadcasted_iota(i32, (8, 128), 0)
    cc = lax.broadcasted_iota(i32, (8, 128), 1)
    stat_ref[...] = (jnp.where((rr == 0) & (cc == 0), mb, 0.0)
                     + jnp.where((rr == 0) & (cc == 1), sb, 0.0))


def _k2a_stats(a2d, pk):
    return pl.pallas_call(
        _k2a_body,
        out_shape=jax.ShapeDtypeStruct((8, 128), f32),
    )(a2d, pk)


BE2 = 4096
NBE2 = EP // BE2


def _k2_body(a_ref, xs_ref, pk_ref, w2_ref, stat_ref, msg_ref):
    i = pl.program_id(0)
    del i
    ab = a_ref[0].reshape(BE2)
    alpha = pk_ref[6, 0]
    beta = pk_ref[6, 1]
    s_l = _lrelu(alpha * ab + beta)
    m0 = stat_ref[0, 0]
    s0 = stat_ref[0, 1]
    sm = jnp.exp(s_l - m0) / s0
    ea = _lrelu(ab * sm)
    A = pk_ref[0]
    C = pk_ref[1]
    h = _lrelu(ea[:, None] * A[None, :] + C[None, :])
    acc = jnp.zeros((BE2, HID), f32)
    for fdim in range(4):
        wf = w2_ref[:, fdim * HID:(fdim + 1) * HID]
        hf = jnp.dot(h, wf, preferred_element_type=f32) + pk_ref[2 + fdim][None, :]
        acc = acc + xs_ref[fdim][:, None] * hf
    msg_ref[...] = acc


def _k2_msg(a3, xs, pk, w2, stats):
    return pl.pallas_call(
        _k2_body,
        grid=(NBE2,),
        in_specs=[
            pl.BlockSpec((1, 32, 128), lambda i: (i, 0, 0)),
            pl.BlockSpec((8, BE2), lambda i: (0, i)),
            pl.BlockSpec((8, HID), lambda i: (0, 0)),
            pl.BlockSpec((HID, NF * HID), lambda i: (0, 0)),
            pl.BlockSpec((8, 128), lambda i: (0, 0)),
        ],
        out_specs=pl.BlockSpec((BE2, HID), lambda i: (i, 0)),
        out_shape=jax.ShapeDtypeStruct((EP, HID), f32),
    )(a3, xs, pk, w2, stats)


CH3 = 512
NCH3 = EPT // CH3


@functools.lru_cache(maxsize=None)
def _build_k3():
    @functools.partial(
        pl.kernel,
        out_type=[
            jax.ShapeDtypeStruct((2, NP, HID), f32),
            jax.ShapeDtypeStruct((2, NP, 16), f32),
        ],
        mesh=_mesh(),
        compiler_params=pltpu.CompilerParams(needs_layout_passes=False, use_tc_tiling_on_sc=False),
        scratch_types=[
            pltpu.VMEM_SHARED((NP, HID), f32),
            pltpu.VMEM_SHARED((NP, 16), f32),
            pltpu.VMEM((4, 128), i32),
            pltpu.VMEM((4, 128), i32),
            pltpu.VMEM((CH3, HID), f32),
            pltpu.VMEM((CH3, HID), f32),
            pltpu.VMEM((128, 16), f32),
            pltpu.SemaphoreType.DMA,
            pltpu.SemaphoreType.DMA,
        ],
    )
    def _k3(msg_hbm, dst2d_hbm, z64_hbm, z16_hbm, ones_hbm,
            aggp_hbm, cntp_hbm, aggS, cntS, dbufA, dbufB, mbufA, mbufB, onesv,
            semA, semB):
        cid = lax.axis_index("c")
        sid = lax.axis_index("s")
        wid = sid * 2 + cid
        rs = pl.multiple_of(sid * NPS, 128)
        pltpu.sync_copy(z64_hbm.at[pl.ds(rs, NPS)], aggS.at[pl.ds(rs, NPS)])
        pltpu.sync_copy(z16_hbm.at[pl.ds(rs, NPS)], cntS.at[pl.ds(rs, NPS)])
        pltpu.sync_copy(ones_hbm, onesv)
        plsc.subcore_barrier()
        dbufs = (dbufA, dbufB)
        mbufs = (mbufA, mbufB)
        sems = (semA, semB)
        hand = [None, None]

        def _issue(ch, b):
            e0 = pl.multiple_of(wid * EPT + ch * CH3, 128)
            r0 = pl.multiple_of(wid * (EPT // 128) + ch * (CH3 // 128), 4)
            h1 = pltpu.async_copy(dst2d_hbm.at[pl.ds(r0, 4)], dbufs[b], sems[b])
            h2 = pltpu.async_copy(msg_hbm.at[pl.ds(e0, CH3)], mbufs[b], sems[b])
            return (h1, h2)

        hand[0] = _issue(0, 0)
        for ch in range(NCH3):
            b = ch % 2
            hand[b][0].wait()
            hand[b][1].wait()
            if ch + 1 < NCH3:
                hand[1 - b] = _issue(ch + 1, 1 - b)
            for r in range(4):
                pltpu.sync_copy(mbufs[b].at[pl.ds(r * 128, 128)],
                                aggS.at[dbufs[b].at[r]], add=True)
                pltpu.sync_copy(onesv, cntS.at[dbufs[b].at[r]], add=True)
        plsc.subcore_barrier()
        pltpu.sync_copy(aggS.at[pl.ds(rs, NPS)], aggp_hbm.at[cid, pl.ds(rs, NPS)])
        pltpu.sync_copy(cntS.at[pl.ds(rs, NPS)], cntp_hbm.at[cid, pl.ds(rs, NPS)])

    return _k3


def _k3_scatter(msg, dst2d, z64, z16, ones16):
    return _build_k3()(msg, dst2d, z64, z16, ones16)


@functools.lru_cache(maxsize=None)
def _build_kseg():
    @functools.partial(
        pl.kernel,
        out_type=jax.ShapeDtypeStruct((2, NP, 16), f32),
        mesh=_mesh(),
        compiler_params=pltpu.CompilerParams(needs_layout_passes=False, use_tc_tiling_on_sc=False),
        scratch_types=[
            pltpu.VMEM_SHARED((NP, 16), f32),
            pltpu.VMEM_SHARED((NP, 16), f32),
            pltpu.VMEM((8, 128), i32),
            pltpu.VMEM((8, 128), i32),
            pltpu.VMEM((128, 16), f32),
            pltpu.VMEM((128, 16), f32),
            pltpu.SemaphoreType.DMA,
            pltpu.SemaphoreType.DMA,
        ],
    )
    def _kseg(y16_hbm, src2d_hbm, dst2d_hbm, z16_hbm, out_hbm,
              ytabS, accS, sbuf, dbuf, gbufA, gbufB, gsemA, gsemB):
        cid = lax.axis_index("c")
        sid = lax.axis_index("s")
        wid = sid * 2 + cid
        rs = pl.multiple_of(sid * NPS, 128)
        pltpu.sync_copy(y16_hbm.at[pl.ds(rs, NPS)], ytabS.at[pl.ds(rs, NPS)])
        pltpu.sync_copy(z16_hbm.at[pl.ds(rs, NPS)], accS.at[pl.ds(rs, NPS)])
        plsc.subcore_barrier()
        gbufs = (gbufA, gbufB)
        gsems = (gsemA, gsemB)
        for ch in range(NCH):
            r0 = pl.multiple_of(wid * (EPT // 128) + ch * (CH // 128), 8)
            pltpu.sync_copy(src2d_hbm.at[pl.ds(r0, 8)], sbuf)
            pltpu.sync_copy(dst2d_hbm.at[pl.ds(r0, 8)], dbuf)
            gh = [None, None]
            gh[0] = pltpu.async_copy(ytabS.at[sbuf.at[0]], gbufs[0], gsems[0])
            for r in range(8):
                b = r % 2
                gh[b].wait()
                if r + 1 < 8:
                    gh[1 - b] = pltpu.async_copy(ytabS.at[sbuf.at[r + 1]],
                                                 gbufs[1 - b], gsems[1 - b])
                pltpu.sync_copy(gbufs[b], accS.at[dbuf.at[r]], add=True)
        plsc.subcore_barrier()
        pltpu.sync_copy(accS.at[pl.ds(rs, NPS)], out_hbm.at[cid, pl.ds(rs, NPS)])

    return _kseg


def _kseg_sum(y16, src2d, dst2d, z16):
    return _build_kseg()(y16, src2d, dst2d, z16)


@functools.lru_cache(maxsize=None)
def _build_k7():
    @functools.partial(
        pl.kernel,
        out_type=[
            jax.ShapeDtypeStruct((2, NP, HID), f32),
            jax.ShapeDtypeStruct((2, NP, 16), f32),
        ],
        mesh=_mesh(),
        compiler_params=pltpu.CompilerParams(needs_layout_passes=False, use_tc_tiling_on_sc=False),
        scratch_types=[
            pltpu.VMEM_SHARED((NP, HID), f32),
            pltpu.VMEM_SHARED((NP, 16), f32),
            pltpu.VMEM((NP * 4,), f32),
            pltpu.VMEM((8, 128), i32),
            pltpu.VMEM((8, 128), i32),
            pltpu.VMEM((CH,), f32),
            pltpu.VMEM((128, 16), f32),
            pltpu.VMEM((128, HID), f32),
            pltpu.VMEM((128, HID), f32),
            pltpu.VMEM((16,), f32),
            pltpu.SemaphoreType.DMA,
            pltpu.SemaphoreType.DMA,
            pltpu.SemaphoreType.DMA,
            pltpu.SemaphoreType.DMA,
        ],
    )
    def _k7(src2d_hbm, dst2d_hbm, ptab_hbm, xt_hbm, m16_hbm, z64_hbm, z16_hbm,
            nump_hbm, denp_hbm,
            numS, denS, ptab, sbuf, dbuf, exbuf, dpay, growA, growB, mv,
            gsemA, gsemB, ssemA, ssemB):
        cid = lax.axis_index("c")
        sid = lax.axis_index("s")
        wid = sid * 2 + cid
        rs = pl.multiple_of(sid * NPS, 128)
        pltpu.sync_copy(ptab_hbm, ptab)
        pltpu.sync_copy(m16_hbm, mv)
        pltpu.sync_copy(z64_hbm.at[pl.ds(rs, NPS)], numS.at[pl.ds(rs, NPS)])
        pltpu.sync_copy(z16_hbm.at[pl.ds(rs, NPS)], denS.at[pl.ds(rs, NPS)])
        pltpu.sync_copy(z16_hbm.at[pl.ds(0, 128)], dpay)
        plsc.subcore_barrier()
        M = mv[...]
        zl = jnp.zeros((16,), i32)
        il = lax.iota(i32, 16)

        def chbody(ch, carry):
            r0 = pl.multiple_of(wid * (EPT // 128) + ch * (CH // 128), 8)
            pltpu.sync_copy(src2d_hbm.at[pl.ds(r0, 8)], sbuf)
            pltpu.sync_copy(dst2d_hbm.at[pl.ds(r0, 8)], dbuf)
            for r in range(8):

                def exbody(j2, carry2, r=r):
                    sv = sbuf[r, pl.ds(j2 * 16, 16)]
                    dv = dbuf[r, pl.ds(j2 * 16, 16)]
                    sv4 = sv * 4
                    dv4 = dv * 4
                    a_s = plsc.load_gather(ptab, [sv4])
                    a_d = plsc.load_gather(ptab, [dv4 + 1])
                    m_s = plsc.load_gather(ptab, [sv4 + 3])
                    m_d = plsc.load_gather(ptab, [dv4 + 3])
                    t = a_s + a_d
                    lg = jnp.where(t >= 0, t, 0.2 * t)
                    ex = jnp.exp(lg - M)
                    exm = jnp.where(m_s * m_d > 0.5, ex, jnp.zeros((16,), f32))
                    exbuf[pl.ds(r * 128 + j2 * 16, 16)] = exm
                    return carry2

                lax.fori_loop(0, 8, exbody, 0)
            bufs = (growA, growB)
            gsems = (gsemA, gsemB)
            ssems = (ssemA, ssemB)
            gh = [None, None]
            sh = [None, None]
            gh[0] = pltpu.async_copy(xt_hbm.at[sbuf.at[0]], bufs[0], gsems[0])
            for r in range(8):
                b = r % 2
                if r + 1 < 8:
                    nb = (r + 1) % 2
                    if r >= 1:
                        sh[nb].wait()
                    gh[nb] = pltpu.async_copy(xt_hbm.at[sbuf.at[r + 1]],
                                              bufs[nb], gsems[nb])
                gh[b].wait()
                for jj in range(8):
                    v = exbuf[pl.ds(r * 128 + jj * 16, 16)]
                    plsc.store_scatter(dpay, [jj * 16 + il, zl], v)
                pltpu.sync_copy(dpay, denS.at[dbuf.at[r]], add=True)
                grow = bufs[b]

                def scbody(g, carry2, r=r, grow=grow):
                    ev = exbuf[pl.ds(r * 128 + g * 16, 16)]
                    for lane in range(16):
                        i = g * 16 + lane
                        e = ev[lane]
                        for q in range(4):
                            grow[i, pl.ds(q * 16, 16)] = e * grow[i, pl.ds(q * 16, 16)]
                    return carry2

                lax.fori_loop(0, 8, scbody, 0)
                sh[b] = pltpu.async_copy(grow, numS.at[dbuf.at[r]], ssems[b], add=True)
            sh[0].wait()
            sh[1].wait()
            return carry

        lax.fori_loop(0, NCH, chbody, 0)
        plsc.subcore_barrier()
        pltpu.sync_copy(numS.at[pl.ds(rs, NPS)], nump_hbm.at[cid, pl.ds(rs, NPS)])
        pltpu.sync_copy(denS.at[pl.ds(rs, NPS)], denp_hbm.at[cid, pl.ds(rs, NPS)])

    return _k7


def _k7_gat(src2d, dst2d, ptab, xt, m16, z64, z16):
    return _build_k7()(src2d, dst2d, ptab.reshape(NP * 4), xt, m16, z64, z16)


def _topk_thresholds(skey2d, k):
    t0 = jnp.where(jnp.sum((skey2d >= 0).astype(i32)) >= k, i32(0), i32(-2**31))

    def bit_body(bi, t):
        b = 30 - bi
        cand = t + (i32(1) << b)
        cnt = jnp.sum((skey2d >= cand).astype(i32))
        return jnp.where(cnt >= k, cand, t)

    t = lax.fori_loop(0, 31, bit_body, t0)
    gt = skey2d > t
    eq = skey2d == t
    need = k - jnp.sum(gt.astype(i32))
    pos = (lax.broadcasted_iota(i32, (80, 128), 0) * 128
           + lax.broadcasted_iota(i32, (80, 128), 1))

    def idx_body(bi, t2):
        b = 13 - bi
        cand = t2 | (i32(1) << b)
        cl = jnp.sum((eq & (pos < cand)).astype(i32))
        return jnp.where(cl < need, cand, t2)

    t2 = lax.fori_loop(0, 14, idx_body, i32(0))
    return t, t2


def _skey(score2d):
    bits = lax.bitcast_convert_type(score2d, i32)
    return jnp.where(bits >= 0, bits, bits ^ i32(0x7FFFFFFF))


def _k4_body(aggp_ref, cntp_ref, x_ref, pk_ref, x1_ref, y16_ref):
    cnt = cntp_ref[0, :, 0:1] + cntp_ref[1, :, 0:1]
    agg = (aggp_ref[0] + aggp_ref[1]) / jnp.maximum(cnt, 1.0)
    root = pk_ref[1:5]
    x1 = agg + jnp.dot(x_ref[...], root, preferred_element_type=f32) + pk_ref[0][None, :]
    x1_ref[...] = x1
    y1 = jnp.dot(x1, pk_ref[5][:, None], preferred_element_type=f32)
    y16_ref[...] = jnp.broadcast_to(y1, (NP, 16))


def _k4(aggp, cntp, xp4, pk):
    return pl.pallas_call(
        _k4_body,
        out_shape=[
            jax.ShapeDtypeStruct((NP, HID), f32),
            jax.ShapeDtypeStruct((NP, 16), f32),
        ],
    )(aggp, cntp, xp4, pk)


def _k6_body(x1_ref, nbsp_ref, pk_ref, gw_ref, xt_ref, ptab_ref, m_ref):
    x1 = x1_ref[...]
    nbs = nbsp_ref[0, :, 0:1] + nbsp_ref[1, :, 0:1]
    score = nbs + pk_ref[5, 0] + jnp.dot(x1, pk_ref[4][:, None], preferred_element_type=f32)
    score2d = score.reshape(80, 128)
    pos = (lax.broadcasted_iota(i32, (80, 128), 0) * 128
           + lax.broadcasted_iota(i32, (80, 128), 1))
    sk = jnp.where(pos < N, _skey(score2d), i32(-2**31))
    t, t2 = _topk_thresholds(sk, K1N)
    posc = lax.broadcasted_iota(i32, (NP, 1), 0)
    skc = jnp.where(posc < N, _skey(score), i32(-2**31))
    m1c = (skc > t) | ((skc == t) & (posc <= t2))
    xp = jnp.maximum(x1 * jnp.tanh(score), 0.0)
    xp = xp * (_INV * pk_ref[2][None, :]) + pk_ref[3][None, :]
    xt = jnp.dot(xp, gw_ref[...], preferred_element_type=f32)
    xt_ref[...] = xt
    asrc = jnp.dot(xt, pk_ref[0][:, None], preferred_element_type=f32)
    adst = jnp.dot(xt, pk_ref[1][:, None], preferred_element_type=f32)
    rowm = posc < N
    Ma = jnp.max(jnp.where(rowm, asrc, -3e38))
    Mb = jnp.max(jnp.where(rowm, adst, -3e38))
    M = _lrelu(Ma + Mb, 0.2)
    ssum = asrc + adst
    lg_s = jnp.where(ssum >= 0, ssum, 0.2 * ssum)
    exs = jnp.exp(lg_s - M)
    ptab_ref[...] = jnp.concatenate([asrc, adst, exs, m1c.astype(f32)], axis=1)
    m_ref[...] = jnp.full((8, 16), M, f32)


def _k6(x1, nbsp, pk, gw):
    return pl.pallas_call(
        _k6_body,
        out_shape=[
            jax.ShapeDtypeStruct((NP, HID), f32),
            jax.ShapeDtypeStruct((NP, 4), f32),
            jax.ShapeDtypeStruct((8, 16), f32),
        ],
    )(x1, nbsp, pk, gw)


def _k8_body(nump_ref, denp_ref, xt_ref, ptab_ref, pk_ref, x2_ref, y16_ref):
    xt = xt_ref[...]
    exs = ptab_ref[:, 2:3]
    m1 = ptab_ref[:, 3:4] > 0.5
    den = denp_ref[0, :, 0:1] + denp_ref[1, :, 0:1] + exs
    num = nump_ref[0] + nump_ref[1] + exs * xt
    x2 = num / den + pk_ref[0][None, :]
    x2 = jnp.where(m1, x2, 0.0)
    x2_ref[...] = x2
    y2 = jnp.dot(x2, pk_ref[1][:, None], preferred_element_type=f32)
    y2m = jnp.where(m1, y2, 0.0)
    y16_ref[...] = jnp.broadcast_to(y2m, (NP, 16))


def _k8(nump, denp, xt, ptab, pk):
    return pl.pallas_call(
        _k8_body,
        out_shape=[
            jax.ShapeDtypeStruct((NP, HID), f32),
            jax.ShapeDtypeStruct((NP, 16), f32),
        ],
    )(nump, denp, xt, ptab, pk)


def _k10_body(x2_ref, nb2p_ref, ptab_ref, pk_ref, wih_ref, whh_ref, sb_ref,
              fcw_ref, fcb_ref, out_ref):
    x2 = x2_ref[...]
    m1 = ptab_ref[:, 3:4] > 0.5
    nb2 = nb2p_ref[0, :, 0:1] + nb2p_ref[1, :, 0:1]
    score = nb2 + pk_ref[3, 0] + jnp.dot(x2, pk_ref[2][:, None], preferred_element_type=f32)
    score2d = score.reshape(80, 128)
    posc = lax.broadcasted_iota(i32, (NP, 1), 0)
    skc = jnp.where((posc < N) & m1, _skey(score), i32(-2**31))
    sk = skc.reshape(80, 128)
    t, t2 = _topk_thresholds(sk, K2N)
    m2 = (skc > t) | ((skc == t) & (posc <= t2))
    x3 = jnp.maximum(x2 * jnp.tanh(score), 0.0)
    x3 = x3 * (_INV * pk_ref[0][None, :]) + pk_ref[1][None, :]

    q_star = jnp.zeros((1, 2 * HID), f32)
    hC = jnp.zeros((1, HID), f32)
    cC = jnp.zeros((1, HID), f32)
    for _ in range(5):
        gates = (jnp.dot(q_star, wih_ref[...], preferred_element_type=f32)
                 + sb_ref[0][None, :]
                 + jnp.dot(hC, whh_ref[...], preferred_element_type=f32)
                 + sb_ref[1][None, :])
        ig = jax.nn.sigmoid(gates[:, 0:HID])
        fg = jax.nn.sigmoid(gates[:, HID:2 * HID])
        gg = jnp.tanh(gates[:, 2 * HID:3 * HID])
        og = jax.nn.sigmoid(gates[:, 3 * HID:4 * HID])
        cC = fg * cC + ig * gg
        hC = og * jnp.tanh(cC)
        eatt = jnp.sum(x3 * hC, axis=1, keepdims=True)
        eatt = jnp.where(m2, eatt, -3e38)
        mx = jnp.max(eatt)
        ex = jnp.where(m2, jnp.exp(eatt - mx), 0.0)
        aw = ex / jnp.sum(ex)
        r = jnp.sum(aw * x3, axis=0, keepdims=True)
        q_star = jnp.concatenate([hC, r], axis=1)

    res = jnp.dot(q_star, fcw_ref[...], preferred_element_type=f32) + fcb_ref[0][None, :]
    out_ref[...] = jnp.zeros((8, 128), f32)
    out_ref[0:1, 0:32] = res


def _k10(x2, nb2p, ptab, pk, wih, whh, sb, fcw, fcb):
    return pl.pallas_call(
        _k10_body,
        out_shape=jax.ShapeDtypeStruct((8, 128), f32),
    )(x2, nb2p, ptab, pk, wih, whh, sb, fcw, fcb)


def kernel(x, edge_attr, params, edge_index, batch):
    p = params
    src = edge_index[0]
    dst = edge_index[1]
    srcp = jnp.concatenate([src, jnp.zeros((EP - E,), i32)])
    dstp = jnp.concatenate([dst, jnp.full((EP - E,), NP - 1, i32)])
    src2d = srcp.reshape(EP // 128, 128)
    dst2d = dstp.reshape(EP // 128, 128)
    apad = jnp.pad(edge_attr[:, 0], (0, EP - E))
    a3 = apad.reshape(NBE2, 32, 128)
    a2d = apad.reshape(EP // 128, 128)
    xp4 = jnp.pad(x, ((0, NP - N), (0, 0)))
    z64 = jnp.zeros((NP, HID), f32)
    z16 = jnp.zeros((NP, 16), f32)
    ones16 = jnp.ones((128, 16), f32)

    alpha = p['ea_et_W'][0] @ p['ea_st_W'][:, 0]
    beta = p['ea_et_b'] @ p['ea_st_W'][:, 0] + p['ea_st_b'][0]
    A = p['em_W1'][0] * _INV * p['em_bn_g']
    C = (p['em_b1'] * _INV) * p['em_bn_g'] + p['em_bn_b']
    B2 = p['em_b2'].reshape(NF, HID)
    pk2 = jnp.zeros((8, HID), f32)
    pk2 = pk2.at[0].set(A).at[1].set(C).at[2:6].set(B2)
    pk2 = pk2.at[6, 0].set(alpha).at[6, 1].set(beta)

    pk4 = jnp.zeros((8, HID), f32)
    pk4 = pk4.at[0].set(p['c1_b']).at[1:5].set(p['c1_root']).at[5].set(p['p1_rel_W'][:, 0])

    pk6 = jnp.zeros((8, HID), f32)
    pk6 = (pk6.at[0].set(p['g_asrc']).at[1].set(p['g_adst'])
              .at[2].set(p['bn1_g']).at[3].set(p['bn1_b'])
              .at[4].set(p['p1_root_W'][:, 0]).at[5, 0].set(p['p1_rel_b'][0]))

    pk8 = jnp.zeros((8, HID), f32)
    pk8 = pk8.at[0].set(p['g_b']).at[1].set(p['p2_rel_W'][:, 0])

    pk10 = jnp.zeros((8, HID), f32)
    pk10 = (pk10.at[0].set(p['bn2_g']).at[1].set(p['bn2_b'])
                .at[2].set(p['p2_root_W'][:, 0]).at[3, 0].set(p['p2_rel_b'][0]))
    wih = p['s2s_Wih'].T
    whh = p['s2s_Whh'].T
    sb = jnp.zeros((8, 4 * HID), f32)
    sb = sb.at[0].set(p['s2s_bih']).at[1].set(p['s2s_bhh'])
    fcw = p['fc_W']
    fcb = jnp.zeros((8, 32), f32).at[0].set(p['fc_b'])

    xs = _k1_gather(xp4.reshape(NP * 4), srcp)
    stats = _k2a_stats(a2d, pk2)
    msg = _k2_msg(a3, xs, pk2, p['em_W2'], stats)
    aggp, cntp = _k3_scatter(msg, dst2d, z64, z16, ones16)
    x1, y16 = _k4(aggp, cntp, xp4, pk4)
    nbsp = _kseg_sum(y16, src2d, dst2d, z16)
    xt, ptab, m8 = _k6(x1, nbsp, pk6, p['g_W'])
    m16 = m8.reshape(128)[0:16]
    nump, denp = _k7_gat(src2d, dst2d, ptab, xt, m16, z64, z16)
    x2, y216 = _k8(nump, denp, xt, ptab, pk8)
    nb2p = _kseg_sum(y216, src2d, dst2d, z16)
    out = _k10(x2, nb2p, ptab, pk10, wih, whh, sb, fcw, fcb)
    return out[0:1, 0:32]
